# Initial kernel scaffold; baseline (speedup 1.0000x reference)
import jax
import jax.numpy as jnp
from jax.experimental import pallas as pl


def kernel(x, x_batch, tgt_y, tgt_edge_index, tgt_edge_type, tgt_y_batch, embed_table, l1_Ws, l1_b, l1_We, l1_Wc, l2_Ws, l2_b, l2_We, l2_Wc, l3_Ws, l3_b, l3_We, l3_Wc, Wz, bz, Wg, bg):
    raise NotImplementedError("write your pallas kernel here")


# speedup vs baseline: 10.2561x; 10.2561x over previous
import functools

import jax
import jax.numpy as jnp
from jax import lax
from jax.experimental import pallas as pl
from jax.experimental.pallas import tpu as pltpu
from jax.experimental.pallas import tpu_sc as plsc

N_X = 20000
N_Y = 10000
E = 320000
B = 16
F_SIZE = 128
H_SIZE = 128
EMB = 128
VOCAB = 1000
T_EDGE = 4
R_EDGE = 9

NC, NS = 2, 16
NW = NC * NS
CH = 80
ZB = 80
NZB = N_Y // ZB

_f32 = jnp.float32


def _sc_mesh():
    return plsc.VectorSubcoreMesh(core_axis_name="c", subcore_axis_name="s")


def _emb_gather(table, idx):
    n_total = idx.shape[0]
    nch = n_total // CH

    @functools.partial(
        pl.kernel, mesh=_sc_mesh(),
        out_type=jax.ShapeDtypeStruct((n_total, 128), _f32),
        scratch_types=[
            pltpu.VMEM((CH,), jnp.int32),
            pltpu.VMEM((CH, 128), _f32),
            pltpu.SemaphoreType.DMA,
        ])
    def k(table_h, idx_h, out_h, idx_v, rows_v, sem):
        c = lax.axis_index("c")
        s = lax.axis_index("s")
        w = s * NC + c
        n = nch // NW + jnp.where(w < (nch % NW), 1, 0)

        def body(t, carry):
            ch = w + NW * t
            base = pl.multiple_of(ch * CH, 8)
            pltpu.sync_copy(idx_h.at[pl.ds(base, CH)], idx_v)
            pltpu.async_copy(table_h.at[idx_v], rows_v, sem).wait()
            pltpu.sync_copy(rows_v, out_h.at[pl.ds(base, CH)])
            return carry

        lax.fori_loop(0, n, body, 0)

    return k(table, idx)


def _edge_agg(hall, src, etype, dst, zeros_z):
    npw = (E // CH) // NW

    @functools.partial(
        pl.kernel, mesh=_sc_mesh(),
        out_type=jax.ShapeDtypeStruct((NC, N_Y, 128), _f32),
        scratch_types=[
            pltpu.VMEM((CH,), jnp.int32),
            pltpu.VMEM((CH,), jnp.int32),
            pltpu.VMEM((CH,), jnp.int32),
            pltpu.VMEM((CH, 128), _f32),
            pltpu.VMEM((ZB, 128), _f32),
            pltpu.VMEM_SHARED((N_Y, 128), _f32),
            pltpu.SemaphoreType.DMA,
        ])
    def k(hall_h, src_h, typ_h, dst_h, z_h, out_h,
          idx_v, typ_v, dst_v, rows_v, zbuf, agg_s, sem):
        c = lax.axis_index("c")
        s = lax.axis_index("s")
        w = s * NC + c

        pltpu.sync_copy(z_h, zbuf)
        nzb_s = NZB // NS + jnp.where(s < (NZB % NS), 1, 0)

        def zbody(t, carry):
            off = pl.multiple_of((s + NS * t) * ZB, 8)
            pltpu.sync_copy(zbuf, agg_s.at[pl.ds(off, ZB)])
            return carry

        lax.fori_loop(0, nzb_s, zbody, 0)
        plsc.subcore_barrier()

        def body(t, carry):
            ch = w * npw + t
            base = pl.multiple_of(ch * CH, 8)
            pltpu.sync_copy(src_h.at[pl.ds(base, CH)], idx_v)
            pltpu.sync_copy(typ_h.at[pl.ds(base, CH)], typ_v)
            pltpu.sync_copy(dst_h.at[pl.ds(base, CH)], dst_v)
            for j in range(CH // 16):
                sl = pl.ds(j * 16, 16)
                idx_v[sl] = idx_v[sl] + typ_v[sl] * N_Y
            pltpu.async_copy(hall_h.at[idx_v], rows_v, sem).wait()
            pltpu.sync_copy(rows_v, agg_s.at[dst_v], add=True)
            return carry

        lax.fori_loop(0, npw, body, 0)
        plsc.subcore_barrier()

        def obody(t, carry):
            off = pl.multiple_of((s + NS * t) * ZB, 8)
            sl = pl.ds(off, ZB)
            pltpu.sync_copy(agg_s.at[sl], out_h.at[c].at[sl])
            return carry

        lax.fori_loop(0, nzb_s, obody, 0)

    return k(hall, src, etype, dst, zeros_z)


def _head_gather(sp, dp, src, dst):
    npw = (E // CH) // NW

    @functools.partial(
        pl.kernel, mesh=_sc_mesh(),
        compiler_params=pltpu.CompilerParams(use_tc_tiling_on_sc=False),
        out_type=(jax.ShapeDtypeStruct((E, 16), _f32),
                  jax.ShapeDtypeStruct((E, 16), _f32)),
        scratch_types=[
            pltpu.VMEM((CH,), jnp.int32),
            pltpu.VMEM((CH,), jnp.int32),
            pltpu.VMEM((CH, 16), _f32),
            pltpu.VMEM((CH, 16), _f32),
            pltpu.SemaphoreType.DMA,
        ])
    def k(sp_h, dp_h, src_h, dst_h, gs_h, gd_h, si_v, di_v, a_v, b_v, sem):
        c = lax.axis_index("c")
        s = lax.axis_index("s")
        w = s * NC + c

        def body(t, carry):
            ch = w * npw + t
            base = pl.multiple_of(ch * CH, 8)
            pltpu.sync_copy(src_h.at[pl.ds(base, CH)], si_v)
            pltpu.sync_copy(dst_h.at[pl.ds(base, CH)], di_v)
            pltpu.async_copy(sp_h.at[si_v], a_v, sem).wait()
            pltpu.async_copy(dp_h.at[di_v], b_v, sem).wait()
            pltpu.sync_copy(a_v, gs_h.at[pl.ds(base, CH)])
            pltpu.sync_copy(b_v, gd_h.at[pl.ds(base, CH)])
            return carry

        lax.fori_loop(0, npw, body, 0)

    return k(sp, dp, src, dst)


_BK = 400


def _ctx_tc(x, x_batch, wc3):
    grid = N_X // _BK

    def kfn(xb_ref, x_ref, wc_ref, out_ref, acc, cnt):
        i = pl.program_id(0)

        @pl.when(i == 0)
        def _():
            acc[...] = jnp.zeros_like(acc)
            cnt[...] = jnp.zeros_like(cnt)

        oh = (xb_ref[...] == lax.broadcasted_iota(jnp.int32, (_BK, B), 1)
              ).astype(_f32)
        acc[...] += lax.dot_general(oh, x_ref[...], (((0,), (0,)), ((), ())),
                                    preferred_element_type=_f32)
        cnt[...] += lax.dot_general(oh, jnp.ones((_BK, 128), _f32),
                                    (((0,), (0,)), ((), ())),
                                    preferred_element_type=_f32)

        @pl.when(i == grid - 1)
        def _():
            ctx = acc[...] / jnp.maximum(cnt[...], 1.0)
            for l in range(3):
                out_ref[l] = jnp.dot(ctx, wc_ref[l],
                                     preferred_element_type=_f32)

    return pl.pallas_call(
        kfn, grid=(grid,),
        in_specs=[pl.BlockSpec((_BK, 1), lambda i: (i, 0)),
                  pl.BlockSpec((_BK, 128), lambda i: (i, 0)),
                  pl.BlockSpec((3, 128, 128), lambda i: (0, 0, 0))],
        out_specs=pl.BlockSpec((3, B, 128), lambda i: (0, 0, 0)),
        out_shape=jax.ShapeDtypeStruct((3, B, 128), _f32),
        scratch_shapes=[pltpu.VMEM((B, 128), _f32),
                        pltpu.VMEM((B, 128), _f32)],
    )(x_batch, x, wc3)


def _pre1_tc(g, wcat):
    grid = N_Y // _BK

    def kfn(g_ref, w_ref, ylin_ref, hall_ref):
        y0 = g_ref[0] + g_ref[1] + g_ref[2] + g_ref[3]
        res = jnp.dot(y0, w_ref[...], preferred_element_type=_f32)
        ylin_ref[...] = res[:, :128]
        for t in range(4):
            hall_ref[t] = res[:, 128 * (t + 1):128 * (t + 2)]

    return pl.pallas_call(
        kfn, grid=(grid,),
        in_specs=[pl.BlockSpec((4, _BK, 128), lambda i: (0, i, 0)),
                  pl.BlockSpec((128, 640), lambda i: (0, 0))],
        out_specs=[pl.BlockSpec((_BK, 128), lambda i: (i, 0)),
                   pl.BlockSpec((4, _BK, 128), lambda i: (0, i, 0))],
        out_shape=[jax.ShapeDtypeStruct((N_Y, 128), _f32),
                   jax.ShapeDtypeStruct((4, N_Y, 128), _f32)],
    )(g, wcat)


def _combine(ylin_ref, agg_ref, ctx_ref, yb_ref, b_ref):
    oh = (yb_ref[...] == lax.broadcasted_iota(jnp.int32, (_BK, B), 1)
          ).astype(_f32)
    ctxg = jnp.dot(oh, ctx_ref[...], preferred_element_type=_f32)
    return jnp.maximum(
        ylin_ref[...] + agg_ref[0] + agg_ref[1] + ctxg + b_ref[...], 0.0)


def _mid_tc(ylin, agg, ctx, yb, bias, wcat):
    grid = N_Y // _BK

    def kfn(ylin_ref, agg_ref, ctx_ref, yb_ref, b_ref, w_ref,
            ylin_o, hall_o):
        y = _combine(ylin_ref, agg_ref, ctx_ref, yb_ref, b_ref)
        res = jnp.dot(y, w_ref[...], preferred_element_type=_f32)
        ylin_o[...] = res[:, :128]
        for t in range(4):
            hall_o[t] = res[:, 128 * (t + 1):128 * (t + 2)]

    return pl.pallas_call(
        kfn, grid=(grid,),
        in_specs=[pl.BlockSpec((_BK, 128), lambda i: (i, 0)),
                  pl.BlockSpec((2, _BK, 128), lambda i: (0, i, 0)),
                  pl.BlockSpec((B, 128), lambda i: (0, 0)),
                  pl.BlockSpec((_BK, 1), lambda i: (i, 0)),
                  pl.BlockSpec((1, 128), lambda i: (0, 0)),
                  pl.BlockSpec((128, 640), lambda i: (0, 0))],
        out_specs=[pl.BlockSpec((_BK, 128), lambda i: (i, 0)),
                   pl.BlockSpec((4, _BK, 128), lambda i: (0, i, 0))],
        out_shape=[jax.ShapeDtypeStruct((N_Y, 128), _f32),
                   jax.ShapeDtypeStruct((4, N_Y, 128), _f32)],
    )(ylin, agg, ctx, yb, bias, wcat)


def _final_tc(ylin, agg, ctx, yb, bias, wz, bz, wgs, wgd, bgs):
    grid = N_Y // _BK

    def kfn(ylin_ref, agg_ref, ctx_ref, yb_ref, b_ref, wz_ref, bz_ref,
            wgs_ref, wgd_ref, bgs_ref, y_o, yp_o, sp_o, dp_o):
        y = _combine(ylin_ref, agg_ref, ctx_ref, yb_ref, b_ref)
        y_o[...] = y
        z = jnp.dot(y, wz_ref[...], preferred_element_type=_f32) + bz_ref[...]
        m = jnp.max(z, axis=1, keepdims=True)
        lse = m + jnp.log(jnp.sum(jnp.exp(z - m), axis=1, keepdims=True))
        yp_o[...] = z - lse
        sp_o[...] = jnp.dot(y, wgs_ref[...],
                            preferred_element_type=_f32) + bgs_ref[...]
        dp_o[...] = jnp.dot(y, wgd_ref[...], preferred_element_type=_f32)

    return pl.pallas_call(
        kfn, grid=(grid,),
        in_specs=[pl.BlockSpec((_BK, 128), lambda i: (i, 0)),
                  pl.BlockSpec((2, _BK, 128), lambda i: (0, i, 0)),
                  pl.BlockSpec((B, 128), lambda i: (0, 0)),
                  pl.BlockSpec((_BK, 1), lambda i: (i, 0)),
                  pl.BlockSpec((1, 128), lambda i: (0, 0)),
                  pl.BlockSpec((128, VOCAB), lambda i: (0, 0)),
                  pl.BlockSpec((1, VOCAB), lambda i: (0, 0)),
                  pl.BlockSpec((128, 16), lambda i: (0, 0)),
                  pl.BlockSpec((128, 16), lambda i: (0, 0)),
                  pl.BlockSpec((1, 16), lambda i: (0, 0))],
        out_specs=[pl.BlockSpec((_BK, 128), lambda i: (i, 0)),
                   pl.BlockSpec((_BK, VOCAB), lambda i: (i, 0)),
                   pl.BlockSpec((_BK, 16), lambda i: (i, 0)),
                   pl.BlockSpec((_BK, 16), lambda i: (i, 0))],
        out_shape=[jax.ShapeDtypeStruct((N_Y, 128), _f32),
                   jax.ShapeDtypeStruct((N_Y, VOCAB), _f32),
                   jax.ShapeDtypeStruct((N_Y, 16), _f32),
                   jax.ShapeDtypeStruct((N_Y, 16), _f32)],
    )(ylin, agg, ctx, yb, bias, wz, bz, wgs, wgd, bgs)


_BKE = 2000


def _edge_softmax_tc(gs, gd):
    grid = E // _BKE

    def kfn(gs_ref, gd_ref, out_ref):
        sv = gs_ref[...] + gd_ref[...]
        m = jnp.max(sv, axis=1, keepdims=True)
        lse = m + jnp.log(jnp.sum(jnp.exp(sv - m), axis=1, keepdims=True))
        out_ref[...] = (sv - lse)[:, :R_EDGE]

    return pl.pallas_call(
        kfn, grid=(grid,),
        in_specs=[pl.BlockSpec((_BKE, 16), lambda i: (i, 0)),
                  pl.BlockSpec((_BKE, 16), lambda i: (i, 0))],
        out_specs=pl.BlockSpec((_BKE, R_EDGE), lambda i: (i, 0)),
        out_shape=jax.ShapeDtypeStruct((E, R_EDGE), _f32),
    )(gs, gd)


def kernel(x, x_batch, tgt_y, tgt_edge_index, tgt_edge_type, tgt_y_batch,
           embed_table,
           l1_Ws, l1_b, l1_We, l1_Wc,
           l2_Ws, l2_b, l2_We, l2_Wc,
           l3_Ws, l3_b, l3_We, l3_Wc,
           Wz, bz, Wg, bg):
    src = tgt_edge_index[0]
    dst = tgt_edge_index[1]
    etype = tgt_edge_type
    yb = tgt_y_batch.reshape(N_Y, 1)
    zeros_z = jnp.zeros((ZB, 128), _f32)

    def wcat(ws, we):
        return jnp.concatenate([ws, we[0], we[1], we[2], we[3]], axis=1)

    neg = jnp.full((7,), -1e30, _f32)
    wgs = jnp.pad(Wg[:EMB], ((0, 0), (0, 7)))
    wgd = jnp.pad(Wg[EMB:], ((0, 0), (0, 7)))
    bgs = jnp.concatenate([bg, neg]).reshape(1, 16)

    g = _emb_gather(embed_table, tgt_y.T.reshape(-1))
    g = g.reshape(4, N_Y, 128)

    ctxp = _ctx_tc(x, x_batch.reshape(N_X, 1), jnp.stack([l1_Wc, l2_Wc, l3_Wc]))

    ylin1, hall1 = _pre1_tc(g, wcat(l1_Ws, l1_We))
    agg1 = _edge_agg(hall1.reshape(4 * N_Y, 128), src, etype, dst, zeros_z)
    ylin2, hall2 = _mid_tc(ylin1, agg1, ctxp[0], yb, l1_b.reshape(1, 128),
                           wcat(l2_Ws, l2_We))
    agg2 = _edge_agg(hall2.reshape(4 * N_Y, 128), src, etype, dst, zeros_z)
    ylin3, hall3 = _mid_tc(ylin2, agg2, ctxp[1], yb, l2_b.reshape(1, 128),
                           wcat(l3_Ws, l3_We))
    agg3 = _edge_agg(hall3.reshape(4 * N_Y, 128), src, etype, dst, zeros_z)

    y, y_pred, sp, dp = _final_tc(ylin3, agg3, ctxp[2], yb,
                                  l3_b.reshape(1, 128), Wz,
                                  bz.reshape(1, VOCAB), wgs, wgd, bgs)

    gs, gd = _head_gather(sp, dp, src, dst)
    y_edge_pred = _edge_softmax_tc(gs, gd)

    return (y, tgt_edge_index, tgt_edge_type, y_pred, y_edge_pred)


# speedup vs baseline: 18.9850x; 1.8511x over previous
import functools

import jax
import jax.numpy as jnp
from jax import lax
from jax.experimental import pallas as pl
from jax.experimental.pallas import tpu as pltpu
from jax.experimental.pallas import tpu_sc as plsc

N_X = 20000
N_Y = 10000
E = 320000
B = 16
F_SIZE = 128
H_SIZE = 128
EMB = 128
VOCAB = 1000
T_EDGE = 4
R_EDGE = 9

NC, NS = 2, 16
NW = NC * NS
CH = 80
ZB = 80
NZB = N_Y // ZB

_f32 = jnp.float32


def _sc_mesh():
    return plsc.VectorSubcoreMesh(core_axis_name="c", subcore_axis_name="s")


def _emb_gather(table, idx):
    n_total = idx.shape[0]
    nch = n_total // CH

    @functools.partial(
        pl.kernel, mesh=_sc_mesh(),
        out_type=jax.ShapeDtypeStruct((n_total, 128), _f32),
        scratch_types=[
            pltpu.VMEM((CH,), jnp.int32),
            pltpu.VMEM((CH, 128), _f32),
            pltpu.SemaphoreType.DMA,
        ])
    def k(table_h, idx_h, out_h, idx_v, rows_v, sem):
        c = lax.axis_index("c")
        s = lax.axis_index("s")
        w = s * NC + c
        n = nch // NW + jnp.where(w < (nch % NW), 1, 0)

        def body(t, carry):
            ch = w + NW * t
            base = pl.multiple_of(ch * CH, 8)
            pltpu.sync_copy(idx_h.at[pl.ds(base, CH)], idx_v)
            pltpu.async_copy(table_h.at[idx_v], rows_v, sem).wait()
            pltpu.sync_copy(rows_v, out_h.at[pl.ds(base, CH)])
            return carry

        lax.fori_loop(0, n, body, 0)

    return k(table, idx)


NPW = (E // CH) // NW


def _edge_agg(hall, flat_idx, dst_idx, zeros_z):
    @functools.partial(
        pl.kernel, mesh=_sc_mesh(),
        compiler_params=pltpu.CompilerParams(use_tc_tiling_on_sc=False),
        out_type=jax.ShapeDtypeStruct((NC, N_Y, 128), _f32),
        scratch_types=[
            pltpu.VMEM((4, CH), jnp.int32),
            pltpu.VMEM((4, CH), jnp.int32),
            pltpu.VMEM((CH, 128), _f32),
            pltpu.VMEM((CH, 128), _f32),
            pltpu.VMEM_SHARED((N_Y, 128), _f32),
            [pltpu.SemaphoreType.DMA] * 4,
            [pltpu.SemaphoreType.DMA] * 2,
            [pltpu.SemaphoreType.DMA] * 2,
        ])
    def k(hall_h, flat_h, dst_h, z_h, out_h,
          fidx, didx, rows0, rows1, agg_s, isems, gsems, ssems):
        c = lax.axis_index("c")
        s = lax.axis_index("s")
        w = s * NC + c
        rows = (rows0, rows1)

        def ibase(j):
            return pl.multiple_of(w * (NPW * CH) + j * CH, 8)

        def prefetch(j, q):
            @pl.when(j < NPW)
            def _():
                sl = pl.ds(ibase(j), CH)
                pltpu.async_copy(flat_h.at[sl], fidx.at[q], isems[q])
                pltpu.async_copy(dst_h.at[sl], didx.at[q], isems[q])

        def iwait(j, q):
            sl = pl.ds(ibase(j), CH)
            pltpu.make_async_copy(flat_h.at[sl], fidx.at[q], isems[q]).wait()
            pltpu.make_async_copy(dst_h.at[sl], didx.at[q], isems[q]).wait()

        for q in range(4):
            prefetch(q, q)

        pltpu.sync_copy(z_h, rows0)
        nzb_s = NZB // NS + jnp.where(s < (NZB % NS), 1, 0)

        def zbody(t, carry):
            off = pl.multiple_of((s + NS * t) * ZB, 8)
            pltpu.sync_copy(rows0, agg_s.at[pl.ds(off, ZB)])
            return carry

        lax.fori_loop(0, nzb_s, zbody, 0)
        plsc.subcore_barrier()

        iwait(0, 0)
        pltpu.async_copy(hall_h.at[fidx.at[0]], rows0, gsems[0])

        def step(j, ph):
            b = ph % 2
            bn = (ph + 1) % 2
            qn = (ph + 1) % 4
            qp = (ph - 1) % 4

            @pl.when(j >= 1)
            def _():
                pltpu.make_async_copy(
                    rows[bn], agg_s.at[didx.at[qp]], ssems[bn]).wait()
                prefetch(j + 3, qp)

            @pl.when(j + 1 < NPW)
            def _():
                iwait(j + 1, qn)
                pltpu.async_copy(hall_h.at[fidx.at[qn]], rows[bn], gsems[bn])

            pltpu.make_async_copy(
                hall_h.at[fidx.at[ph]], rows[b], gsems[b]).wait()
            pltpu.async_copy(rows[b], agg_s.at[didx.at[ph]], ssems[b],
                             add=True)

        def body(i, carry):
            for ph in range(4):
                step(4 * i + ph, ph)
            return carry

        lax.fori_loop(0, NPW // 4, body, 0)
        step(NPW - 1, (NPW - 1) % 4)
        pltpu.make_async_copy(
            rows[(NPW - 1) % 2],
            agg_s.at[didx.at[(NPW - 1) % 4]], ssems[(NPW - 1) % 2]).wait()

        plsc.subcore_barrier()

        def obody(t, carry):
            off = pl.multiple_of((s + NS * t) * ZB, 8)
            sl = pl.ds(off, ZB)
            pltpu.sync_copy(agg_s.at[sl], out_h.at[c].at[sl])
            return carry

        lax.fori_loop(0, nzb_s, obody, 0)

    return k(hall, flat_idx, dst_idx, zeros_z)


def _head_gather(sp, dp, src3, dst3):
    @functools.partial(
        pl.kernel, mesh=_sc_mesh(),
        compiler_params=pltpu.CompilerParams(use_tc_tiling_on_sc=False),
        out_type=(jax.ShapeDtypeStruct((E, 16), _f32),
                  jax.ShapeDtypeStruct((E, 16), _f32)),
        scratch_types=[
            pltpu.VMEM((NPW, CH), jnp.int32),
            pltpu.VMEM((NPW, CH), jnp.int32),
            pltpu.VMEM((CH, 16), _f32),
            pltpu.VMEM((CH, 16), _f32),
            pltpu.VMEM((CH, 16), _f32),
            pltpu.VMEM((CH, 16), _f32),
            pltpu.SemaphoreType.DMA,
            pltpu.SemaphoreType.DMA,
            pltpu.SemaphoreType.DMA,
            pltpu.SemaphoreType.DMA,
            pltpu.SemaphoreType.DMA,
            pltpu.SemaphoreType.DMA,
            pltpu.SemaphoreType.DMA,
            pltpu.SemaphoreType.DMA,
            pltpu.SemaphoreType.DMA,
        ])
    def k(sp_h, dp_h, src_h, dst_h, gs_h, gd_h, si_v, di_v,
          a0, a1, b0, b1, ga0, ga1, gb0, gb1, wa0, wa1, wb0, wb1, msem):
        c = lax.axis_index("c")
        s = lax.axis_index("s")
        w = s * NC + c

        pltpu.async_copy(src_h.at[w], si_v, msem)
        pltpu.async_copy(dst_h.at[w], di_v, msem)
        pltpu.make_async_copy(src_h.at[w], si_v, msem).wait()
        pltpu.make_async_copy(dst_h.at[w], di_v, msem).wait()

        bufs = ((a0, ga0, wa0, b0, gb0, wb0), (a1, ga1, wa1, b1, gb1, wb1))

        def gissue(j, av, gas, bv, gbs):
            pltpu.async_copy(sp_h.at[si_v.at[j]], av, gas)
            pltpu.async_copy(dp_h.at[di_v.at[j]], bv, gbs)

        gissue(0, bufs[0][0], bufs[0][1], bufs[0][3], bufs[0][4])
        gissue(1, bufs[1][0], bufs[1][1], bufs[1][3], bufs[1][4])

        def step(j, av, gas, was, bv, gbs, wbs):
            base = pl.multiple_of((w * NPW + j) * CH, 8)
            osl = pl.ds(base, CH)
            pltpu.make_async_copy(sp_h.at[si_v.at[j]], av, gas).wait()
            pltpu.async_copy(av, gs_h.at[osl], was)
            pltpu.make_async_copy(dp_h.at[di_v.at[j]], bv, gbs).wait()
            pltpu.async_copy(bv, gd_h.at[osl], wbs)
            pltpu.make_async_copy(av, gs_h.at[osl], was).wait()
            pltpu.make_async_copy(bv, gd_h.at[osl], wbs).wait()

            @pl.when(j + 2 < NPW)
            def _():
                gissue(j + 2, av, gas, bv, gbs)

        def body(i, carry):
            for b, bb in enumerate(bufs):
                step(2 * i + b, *bb)
            return carry

        lax.fori_loop(0, NPW // 2, body, 0)
        step(NPW - 1, *bufs[(NPW - 1) % 2])

    return k(sp, dp, src3, dst3)


_BK = 400


def _flat_idx_tc(src2, typ2):
    def kfn(s_ref, t_ref, o_ref):
        o_ref[...] = t_ref[...] * N_Y + s_ref[...]

    return pl.pallas_call(
        kfn,
        out_shape=jax.ShapeDtypeStruct((E // 128, 128), jnp.int32),
    )(src2, typ2)


def _ctx_tc(x, x_batch, wc3):
    grid = N_X // _BK

    def kfn(xb_ref, x_ref, wc_ref, out_ref, acc, cnt):
        i = pl.program_id(0)

        @pl.when(i == 0)
        def _():
            acc[...] = jnp.zeros_like(acc)
            cnt[...] = jnp.zeros_like(cnt)

        oh = (xb_ref[...] == lax.broadcasted_iota(jnp.int32, (_BK, B), 1)
              ).astype(_f32)
        acc[...] += lax.dot_general(oh, x_ref[...], (((0,), (0,)), ((), ())),
                                    preferred_element_type=_f32)
        cnt[...] += lax.dot_general(oh, jnp.ones((_BK, 128), _f32),
                                    (((0,), (0,)), ((), ())),
                                    preferred_element_type=_f32)

        @pl.when(i == grid - 1)
        def _():
            ctx = acc[...] / jnp.maximum(cnt[...], 1.0)
            for l in range(3):
                out_ref[l] = jnp.dot(ctx, wc_ref[l],
                                     preferred_element_type=_f32)

    return pl.pallas_call(
        kfn, grid=(grid,),
        in_specs=[pl.BlockSpec((_BK, 1), lambda i: (i, 0)),
                  pl.BlockSpec((_BK, 128), lambda i: (i, 0)),
                  pl.BlockSpec((3, 128, 128), lambda i: (0, 0, 0))],
        out_specs=pl.BlockSpec((3, B, 128), lambda i: (0, 0, 0)),
        out_shape=jax.ShapeDtypeStruct((3, B, 128), _f32),
        scratch_shapes=[pltpu.VMEM((B, 128), _f32),
                        pltpu.VMEM((B, 128), _f32)],
    )(x_batch, x, wc3)


def _pre1_tc(g, wcat):
    grid = N_Y // _BK

    def kfn(g_ref, w_ref, ylin_ref, hall_ref):
        y0 = g_ref[0] + g_ref[1] + g_ref[2] + g_ref[3]
        res = jnp.dot(y0, w_ref[...], preferred_element_type=_f32)
        ylin_ref[...] = res[:, :128]
        for t in range(4):
            hall_ref[t] = res[:, 128 * (t + 1):128 * (t + 2)]

    return pl.pallas_call(
        kfn, grid=(grid,),
        in_specs=[pl.BlockSpec((4, _BK, 128), lambda i: (0, i, 0)),
                  pl.BlockSpec((128, 640), lambda i: (0, 0))],
        out_specs=[pl.BlockSpec((_BK, 128), lambda i: (i, 0)),
                   pl.BlockSpec((4, _BK, 128), lambda i: (0, i, 0))],
        out_shape=[jax.ShapeDtypeStruct((N_Y, 128), _f32),
                   jax.ShapeDtypeStruct((4, N_Y, 128), _f32)],
    )(g, wcat)


def _combine(ylin_ref, agg_ref, ctx_ref, yb_ref, b_ref):
    oh = (yb_ref[...] == lax.broadcasted_iota(jnp.int32, (_BK, B), 1)
          ).astype(_f32)
    ctxg = jnp.dot(oh, ctx_ref[...], preferred_element_type=_f32)
    return jnp.maximum(
        ylin_ref[...] + agg_ref[0] + agg_ref[1] + ctxg + b_ref[...], 0.0)


def _mid_tc(ylin, agg, ctx, yb, bias, wcat):
    grid = N_Y // _BK

    def kfn(ylin_ref, agg_ref, ctx_ref, yb_ref, b_ref, w_ref,
            ylin_o, hall_o):
        y = _combine(ylin_ref, agg_ref, ctx_ref, yb_ref, b_ref)
        res = jnp.dot(y, w_ref[...], preferred_element_type=_f32)
        ylin_o[...] = res[:, :128]
        for t in range(4):
            hall_o[t] = res[:, 128 * (t + 1):128 * (t + 2)]

    return pl.pallas_call(
        kfn, grid=(grid,),
        in_specs=[pl.BlockSpec((_BK, 128), lambda i: (i, 0)),
                  pl.BlockSpec((2, _BK, 128), lambda i: (0, i, 0)),
                  pl.BlockSpec((B, 128), lambda i: (0, 0)),
                  pl.BlockSpec((_BK, 1), lambda i: (i, 0)),
                  pl.BlockSpec((1, 128), lambda i: (0, 0)),
                  pl.BlockSpec((128, 640), lambda i: (0, 0))],
        out_specs=[pl.BlockSpec((_BK, 128), lambda i: (i, 0)),
                   pl.BlockSpec((4, _BK, 128), lambda i: (0, i, 0))],
        out_shape=[jax.ShapeDtypeStruct((N_Y, 128), _f32),
                   jax.ShapeDtypeStruct((4, N_Y, 128), _f32)],
    )(ylin, agg, ctx, yb, bias, wcat)


def _final_tc(ylin, agg, ctx, yb, bias, wz, bz, wgs, wgd, bgs):
    grid = N_Y // _BK

    def kfn(ylin_ref, agg_ref, ctx_ref, yb_ref, b_ref, wz_ref, bz_ref,
            wgs_ref, wgd_ref, bgs_ref, y_o, yp_o, sp_o, dp_o):
        y = _combine(ylin_ref, agg_ref, ctx_ref, yb_ref, b_ref)
        y_o[...] = y
        z = jnp.dot(y, wz_ref[...], preferred_element_type=_f32) + bz_ref[...]
        m = jnp.max(z, axis=1, keepdims=True)
        lse = m + jnp.log(jnp.sum(jnp.exp(z - m), axis=1, keepdims=True))
        yp_o[...] = z - lse
        sp_o[...] = jnp.dot(y, wgs_ref[...],
                            preferred_element_type=_f32) + bgs_ref[...]
        dp_o[...] = jnp.dot(y, wgd_ref[...], preferred_element_type=_f32)

    return pl.pallas_call(
        kfn, grid=(grid,),
        in_specs=[pl.BlockSpec((_BK, 128), lambda i: (i, 0)),
                  pl.BlockSpec((2, _BK, 128), lambda i: (0, i, 0)),
                  pl.BlockSpec((B, 128), lambda i: (0, 0)),
                  pl.BlockSpec((_BK, 1), lambda i: (i, 0)),
                  pl.BlockSpec((1, 128), lambda i: (0, 0)),
                  pl.BlockSpec((128, VOCAB), lambda i: (0, 0)),
                  pl.BlockSpec((1, VOCAB), lambda i: (0, 0)),
                  pl.BlockSpec((128, 16), lambda i: (0, 0)),
                  pl.BlockSpec((128, 16), lambda i: (0, 0)),
                  pl.BlockSpec((1, 16), lambda i: (0, 0))],
        out_specs=[pl.BlockSpec((_BK, 128), lambda i: (i, 0)),
                   pl.BlockSpec((_BK, VOCAB), lambda i: (i, 0)),
                   pl.BlockSpec((_BK, 16), lambda i: (i, 0)),
                   pl.BlockSpec((_BK, 16), lambda i: (i, 0))],
        out_shape=[jax.ShapeDtypeStruct((N_Y, 128), _f32),
                   jax.ShapeDtypeStruct((N_Y, VOCAB), _f32),
                   jax.ShapeDtypeStruct((N_Y, 16), _f32),
                   jax.ShapeDtypeStruct((N_Y, 16), _f32)],
    )(ylin, agg, ctx, yb, bias, wz, bz, wgs, wgd, bgs)


_BKE = 2000


def _edge_softmax_tc(gs, gd):
    grid = E // _BKE

    def kfn(gs_ref, gd_ref, out_ref):
        sv = gs_ref[...] + gd_ref[...]
        m = jnp.max(sv, axis=1, keepdims=True)
        lse = m + jnp.log(jnp.sum(jnp.exp(sv - m), axis=1, keepdims=True))
        out_ref[...] = (sv - lse)[:, :R_EDGE]

    return pl.pallas_call(
        kfn, grid=(grid,),
        in_specs=[pl.BlockSpec((_BKE, 16), lambda i: (i, 0)),
                  pl.BlockSpec((_BKE, 16), lambda i: (i, 0))],
        out_specs=pl.BlockSpec((_BKE, R_EDGE), lambda i: (i, 0)),
        out_shape=jax.ShapeDtypeStruct((E, R_EDGE), _f32),
    )(gs, gd)


def kernel(x, x_batch, tgt_y, tgt_edge_index, tgt_edge_type, tgt_y_batch,
           embed_table,
           l1_Ws, l1_b, l1_We, l1_Wc,
           l2_Ws, l2_b, l2_We, l2_Wc,
           l3_Ws, l3_b, l3_We, l3_Wc,
           Wz, bz, Wg, bg):
    src3 = tgt_edge_index[0].reshape(NW, NPW, CH)
    dst3 = tgt_edge_index[1].reshape(NW, NPW, CH)
    dst1 = tgt_edge_index[1]
    flat1 = _flat_idx_tc(tgt_edge_index[0].reshape(E // 128, 128),
                         tgt_edge_type.reshape(E // 128, 128)).reshape(E)
    yb = tgt_y_batch.reshape(N_Y, 1)
    zeros_z = jnp.zeros((ZB, 128), _f32)

    def wcat(ws, we):
        return jnp.concatenate([ws, we[0], we[1], we[2], we[3]], axis=1)

    neg = jnp.full((7,), -1e30, _f32)
    wgs = jnp.pad(Wg[:EMB], ((0, 0), (0, 7)))
    wgd = jnp.pad(Wg[EMB:], ((0, 0), (0, 7)))
    bgs = jnp.concatenate([bg, neg]).reshape(1, 16)

    g = _emb_gather(embed_table, tgt_y.T.reshape(-1))
    g = g.reshape(4, N_Y, 128)

    ctxp = _ctx_tc(x, x_batch.reshape(N_X, 1), jnp.stack([l1_Wc, l2_Wc, l3_Wc]))

    ylin1, hall1 = _pre1_tc(g, wcat(l1_Ws, l1_We))
    agg1 = _edge_agg(hall1.reshape(4 * N_Y, 128), flat1, dst1, zeros_z)
    ylin2, hall2 = _mid_tc(ylin1, agg1, ctxp[0], yb, l1_b.reshape(1, 128),
                           wcat(l2_Ws, l2_We))
    agg2 = _edge_agg(hall2.reshape(4 * N_Y, 128), flat1, dst1, zeros_z)
    ylin3, hall3 = _mid_tc(ylin2, agg2, ctxp[1], yb, l2_b.reshape(1, 128),
                           wcat(l3_Ws, l3_We))
    agg3 = _edge_agg(hall3.reshape(4 * N_Y, 128), flat1, dst1, zeros_z)

    y, y_pred, sp, dp = _final_tc(ylin3, agg3, ctxp[2], yb,
                                  l3_b.reshape(1, 128), Wz,
                                  bz.reshape(1, VOCAB), wgs, wgd, bgs)

    gs, gd = _head_gather(sp, dp, src3, dst3)
    y_edge_pred = _edge_softmax_tc(gs, gd)

    return (y, tgt_edge_index, tgt_edge_type, y_pred, y_edge_pred)


# speedup vs baseline: 24.8215x; 1.3074x over previous
import functools

import jax
import jax.numpy as jnp
from jax import lax
from jax.experimental import pallas as pl
from jax.experimental.pallas import tpu as pltpu
from jax.experimental.pallas import tpu_sc as plsc

N_X = 20000
N_Y = 10000
E = 320000
B = 16
F_SIZE = 128
H_SIZE = 128
EMB = 128
VOCAB = 1000
T_EDGE = 4
R_EDGE = 9

NC, NS = 2, 16
NW = NC * NS
CH = 80
ZB = 80
NZB = N_Y // ZB

_f32 = jnp.float32


def _sc_mesh():
    return plsc.VectorSubcoreMesh(core_axis_name="c", subcore_axis_name="s")


def _emb_gather(table, idx):
    n_total = idx.shape[0]
    nch = n_total // CH

    @functools.partial(
        pl.kernel, mesh=_sc_mesh(),
        out_type=jax.ShapeDtypeStruct((n_total, 128), _f32),
        scratch_types=[
            pltpu.VMEM((CH,), jnp.int32),
            pltpu.VMEM((CH, 128), _f32),
            pltpu.SemaphoreType.DMA,
        ])
    def k(table_h, idx_h, out_h, idx_v, rows_v, sem):
        c = lax.axis_index("c")
        s = lax.axis_index("s")
        w = s * NC + c
        n = nch // NW + jnp.where(w < (nch % NW), 1, 0)

        def body(t, carry):
            ch = w + NW * t
            base = pl.multiple_of(ch * CH, 8)
            pltpu.sync_copy(idx_h.at[pl.ds(base, CH)], idx_v)
            pltpu.async_copy(table_h.at[idx_v], rows_v, sem).wait()
            pltpu.sync_copy(rows_v, out_h.at[pl.ds(base, CH)])
            return carry

        lax.fori_loop(0, n, body, 0)

    return k(table, idx)


NPW = (E // CH) // NW


def _edge_agg(hall, flat_idx, dst_idx, zeros_z):
    @functools.partial(
        pl.kernel, mesh=_sc_mesh(),
        compiler_params=pltpu.CompilerParams(use_tc_tiling_on_sc=False),
        out_type=jax.ShapeDtypeStruct((NC, N_Y, 128), _f32),
        scratch_types=[
            pltpu.VMEM((4, CH), jnp.int32),
            pltpu.VMEM((4, CH), jnp.int32),
            pltpu.VMEM((CH, 128), _f32),
            pltpu.VMEM((CH, 128), _f32),
            pltpu.VMEM_SHARED((N_Y, 128), _f32),
            [pltpu.SemaphoreType.DMA] * 4,
            [pltpu.SemaphoreType.DMA] * 2,
            [pltpu.SemaphoreType.DMA] * 2,
        ])
    def k(hall_h, flat_h, dst_h, z_h, out_h,
          fidx, didx, rows0, rows1, agg_s, isems, gsems, ssems):
        c = lax.axis_index("c")
        s = lax.axis_index("s")
        w = s * NC + c
        rows = (rows0, rows1)

        def ibase(j):
            return pl.multiple_of(w * (NPW * CH) + j * CH, 8)

        def prefetch(j, q):
            @pl.when(j < NPW)
            def _():
                sl = pl.ds(ibase(j), CH)
                pltpu.async_copy(flat_h.at[sl], fidx.at[q], isems[q])
                pltpu.async_copy(dst_h.at[sl], didx.at[q], isems[q])

        def iwait(j, q):
            sl = pl.ds(ibase(j), CH)
            pltpu.make_async_copy(flat_h.at[sl], fidx.at[q], isems[q]).wait()
            pltpu.make_async_copy(dst_h.at[sl], didx.at[q], isems[q]).wait()

        for q in range(4):
            prefetch(q, q)

        pltpu.sync_copy(z_h, rows0)
        nzb_s = NZB // NS + jnp.where(s < (NZB % NS), 1, 0)

        def zbody(t, carry):
            off = pl.multiple_of((s + NS * t) * ZB, 8)
            pltpu.sync_copy(rows0, agg_s.at[pl.ds(off, ZB)])
            return carry

        lax.fori_loop(0, nzb_s, zbody, 0)
        plsc.subcore_barrier()

        iwait(0, 0)
        pltpu.async_copy(hall_h.at[fidx.at[0]], rows0, gsems[0])

        def step(j, ph):
            b = ph % 2
            bn = (ph + 1) % 2
            qn = (ph + 1) % 4
            qp = (ph - 1) % 4

            @pl.when(j >= 1)
            def _():
                pltpu.make_async_copy(
                    rows[bn], agg_s.at[didx.at[qp]], ssems[bn]).wait()
                prefetch(j + 3, qp)

            @pl.when(j + 1 < NPW)
            def _():
                iwait(j + 1, qn)
                pltpu.async_copy(hall_h.at[fidx.at[qn]], rows[bn], gsems[bn])

            pltpu.make_async_copy(
                hall_h.at[fidx.at[ph]], rows[b], gsems[b]).wait()
            pltpu.async_copy(rows[b], agg_s.at[didx.at[ph]], ssems[b],
                             add=True)

        def body(i, carry):
            for ph in range(4):
                step(4 * i + ph, ph)
            return carry

        lax.fori_loop(0, NPW // 4, body, 0)
        step(NPW - 1, (NPW - 1) % 4)
        pltpu.make_async_copy(
            rows[(NPW - 1) % 2],
            agg_s.at[didx.at[(NPW - 1) % 4]], ssems[(NPW - 1) % 2]).wait()

        plsc.subcore_barrier()

        def obody(t, carry):
            off = pl.multiple_of((s + NS * t) * ZB, 8)
            sl = pl.ds(off, ZB)
            pltpu.sync_copy(agg_s.at[sl], out_h.at[c].at[sl])
            return carry

        lax.fori_loop(0, nzb_s, obody, 0)

    return k(hall, flat_idx, dst_idx, zeros_z)


def _head_gather(sp, dp, src3, dst3):
    @functools.partial(
        pl.kernel, mesh=_sc_mesh(),
        compiler_params=pltpu.CompilerParams(use_tc_tiling_on_sc=False),
        out_type=(jax.ShapeDtypeStruct((E, 16), _f32),
                  jax.ShapeDtypeStruct((E, 16), _f32)),
        scratch_types=[
            pltpu.VMEM((NPW, CH), jnp.int32),
            pltpu.VMEM((NPW, CH), jnp.int32),
            pltpu.VMEM((CH, 16), _f32),
            pltpu.VMEM((CH, 16), _f32),
            pltpu.VMEM((CH, 16), _f32),
            pltpu.VMEM((CH, 16), _f32),
            pltpu.SemaphoreType.DMA,
            pltpu.SemaphoreType.DMA,
            pltpu.SemaphoreType.DMA,
            pltpu.SemaphoreType.DMA,
            pltpu.SemaphoreType.DMA,
            pltpu.SemaphoreType.DMA,
            pltpu.SemaphoreType.DMA,
            pltpu.SemaphoreType.DMA,
            pltpu.SemaphoreType.DMA,
        ])
    def k(sp_h, dp_h, src_h, dst_h, gs_h, gd_h, si_v, di_v,
          a0, a1, b0, b1, ga0, ga1, gb0, gb1, wa0, wa1, wb0, wb1, msem):
        c = lax.axis_index("c")
        s = lax.axis_index("s")
        w = s * NC + c

        pltpu.async_copy(src_h.at[w], si_v, msem)
        pltpu.async_copy(dst_h.at[w], di_v, msem)
        pltpu.make_async_copy(src_h.at[w], si_v, msem).wait()
        pltpu.make_async_copy(dst_h.at[w], di_v, msem).wait()

        bufs = ((a0, ga0, wa0, b0, gb0, wb0), (a1, ga1, wa1, b1, gb1, wb1))

        def gissue(j, av, gas, bv, gbs):
            pltpu.async_copy(sp_h.at[si_v.at[j]], av, gas)
            pltpu.async_copy(dp_h.at[di_v.at[j]], bv, gbs)

        gissue(0, bufs[0][0], bufs[0][1], bufs[0][3], bufs[0][4])
        gissue(1, bufs[1][0], bufs[1][1], bufs[1][3], bufs[1][4])

        def step(j, av, gas, was, bv, gbs, wbs):
            base = pl.multiple_of((w * NPW + j) * CH, 8)
            osl = pl.ds(base, CH)
            pltpu.make_async_copy(sp_h.at[si_v.at[j]], av, gas).wait()
            pltpu.async_copy(av, gs_h.at[osl], was)
            pltpu.make_async_copy(dp_h.at[di_v.at[j]], bv, gbs).wait()
            pltpu.async_copy(bv, gd_h.at[osl], wbs)
            pltpu.make_async_copy(av, gs_h.at[osl], was).wait()
            pltpu.make_async_copy(bv, gd_h.at[osl], wbs).wait()

            @pl.when(j + 2 < NPW)
            def _():
                gissue(j + 2, av, gas, bv, gbs)

        def body(i, carry):
            for b, bb in enumerate(bufs):
                step(2 * i + b, *bb)
            return carry

        lax.fori_loop(0, NPW // 2, body, 0)
        step(NPW - 1, *bufs[(NPW - 1) % 2])

    return k(sp, dp, src3, dst3)


_BK = 400


def _flat_idx_tc(src2, typ2):
    def kfn(s_ref, t_ref, o_ref):
        o_ref[...] = t_ref[...] * N_Y + s_ref[...]

    return pl.pallas_call(
        kfn,
        out_shape=jax.ShapeDtypeStruct((E // 128, 128), jnp.int32),
    )(src2, typ2)


def _ctx_tc(x, x_batch, wc3):
    grid = N_X // _BK

    def kfn(xb_ref, x_ref, wc_ref, out_ref, acc, cnt):
        i = pl.program_id(0)

        @pl.when(i == 0)
        def _():
            acc[...] = jnp.zeros_like(acc)
            cnt[...] = jnp.zeros_like(cnt)

        oh = (xb_ref[...] == lax.broadcasted_iota(jnp.int32, (_BK, B), 1)
              ).astype(_f32)
        acc[...] += lax.dot_general(oh, x_ref[...], (((0,), (0,)), ((), ())),
                                    preferred_element_type=_f32)
        cnt[...] += lax.dot_general(oh, jnp.ones((_BK, 128), _f32),
                                    (((0,), (0,)), ((), ())),
                                    preferred_element_type=_f32)

        @pl.when(i == grid - 1)
        def _():
            ctx = acc[...] / jnp.maximum(cnt[...], 1.0)
            for l in range(3):
                out_ref[l] = jnp.dot(ctx, wc_ref[l],
                                     preferred_element_type=_f32)

    return pl.pallas_call(
        kfn, grid=(grid,),
        in_specs=[pl.BlockSpec((_BK, 1), lambda i: (i, 0)),
                  pl.BlockSpec((_BK, 128), lambda i: (i, 0)),
                  pl.BlockSpec((3, 128, 128), lambda i: (0, 0, 0))],
        out_specs=pl.BlockSpec((3, B, 128), lambda i: (0, 0, 0)),
        out_shape=jax.ShapeDtypeStruct((3, B, 128), _f32),
        scratch_shapes=[pltpu.VMEM((B, 128), _f32),
                        pltpu.VMEM((B, 128), _f32)],
    )(x_batch, x, wc3)


def _pre1_tc(g, wcat):
    grid = N_Y // _BK

    def kfn(g_ref, w_ref, ylin_ref, hall_ref):
        y0 = g_ref[0] + g_ref[1] + g_ref[2] + g_ref[3]
        res = jnp.dot(y0, w_ref[...], preferred_element_type=_f32)
        ylin_ref[...] = res[:, :128]
        for t in range(4):
            hall_ref[t] = res[:, 128 * (t + 1):128 * (t + 2)]

    return pl.pallas_call(
        kfn, grid=(grid,),
        in_specs=[pl.BlockSpec((4, _BK, 128), lambda i: (0, i, 0)),
                  pl.BlockSpec((128, 640), lambda i: (0, 0))],
        out_specs=[pl.BlockSpec((_BK, 128), lambda i: (i, 0)),
                   pl.BlockSpec((4, _BK, 128), lambda i: (0, i, 0))],
        out_shape=[jax.ShapeDtypeStruct((N_Y, 128), _f32),
                   jax.ShapeDtypeStruct((4, N_Y, 128), _f32)],
    )(g, wcat)


def _combine(ylin_ref, agg_ref, ctx_ref, yb_ref, b_ref):
    oh = (yb_ref[...] == lax.broadcasted_iota(jnp.int32, (_BK, B), 1)
          ).astype(_f32)
    ctxg = jnp.dot(oh, ctx_ref[...], preferred_element_type=_f32)
    return jnp.maximum(
        ylin_ref[...] + agg_ref[0] + agg_ref[1] + ctxg + b_ref[...], 0.0)


def _mid_tc(ylin, agg, ctx, yb, bias, wcat):
    grid = N_Y // _BK

    def kfn(ylin_ref, agg_ref, ctx_ref, yb_ref, b_ref, w_ref,
            ylin_o, hall_o):
        y = _combine(ylin_ref, agg_ref, ctx_ref, yb_ref, b_ref)
        res = jnp.dot(y, w_ref[...], preferred_element_type=_f32)
        ylin_o[...] = res[:, :128]
        for t in range(4):
            hall_o[t] = res[:, 128 * (t + 1):128 * (t + 2)]

    return pl.pallas_call(
        kfn, grid=(grid,),
        in_specs=[pl.BlockSpec((_BK, 128), lambda i: (i, 0)),
                  pl.BlockSpec((2, _BK, 128), lambda i: (0, i, 0)),
                  pl.BlockSpec((B, 128), lambda i: (0, 0)),
                  pl.BlockSpec((_BK, 1), lambda i: (i, 0)),
                  pl.BlockSpec((1, 128), lambda i: (0, 0)),
                  pl.BlockSpec((128, 640), lambda i: (0, 0))],
        out_specs=[pl.BlockSpec((_BK, 128), lambda i: (i, 0)),
                   pl.BlockSpec((4, _BK, 128), lambda i: (0, i, 0))],
        out_shape=[jax.ShapeDtypeStruct((N_Y, 128), _f32),
                   jax.ShapeDtypeStruct((4, N_Y, 128), _f32)],
    )(ylin, agg, ctx, yb, bias, wcat)


def _final_tc(ylin, agg, ctx, yb, bias, wz, bz, wgs, wgd, bgs):
    grid = N_Y // _BK

    def kfn(ylin_ref, agg_ref, ctx_ref, yb_ref, b_ref, wz_ref, bz_ref,
            wgs_ref, wgd_ref, bgs_ref, y_o, yp_o, sp_o, dp_o):
        y = _combine(ylin_ref, agg_ref, ctx_ref, yb_ref, b_ref)
        y_o[...] = y
        z = jnp.dot(y, wz_ref[...], preferred_element_type=_f32) + bz_ref[...]
        m = jnp.max(z, axis=1, keepdims=True)
        lse = m + jnp.log(jnp.sum(jnp.exp(z - m), axis=1, keepdims=True))
        yp_o[...] = z - lse
        sp_o[...] = jnp.dot(y, wgs_ref[...],
                            preferred_element_type=_f32) + bgs_ref[...]
        dp_o[...] = jnp.dot(y, wgd_ref[...], preferred_element_type=_f32)

    return pl.pallas_call(
        kfn, grid=(grid,),
        in_specs=[pl.BlockSpec((_BK, 128), lambda i: (i, 0)),
                  pl.BlockSpec((2, _BK, 128), lambda i: (0, i, 0)),
                  pl.BlockSpec((B, 128), lambda i: (0, 0)),
                  pl.BlockSpec((_BK, 1), lambda i: (i, 0)),
                  pl.BlockSpec((1, 128), lambda i: (0, 0)),
                  pl.BlockSpec((128, VOCAB), lambda i: (0, 0)),
                  pl.BlockSpec((1, VOCAB), lambda i: (0, 0)),
                  pl.BlockSpec((128, 16), lambda i: (0, 0)),
                  pl.BlockSpec((128, 16), lambda i: (0, 0)),
                  pl.BlockSpec((1, 16), lambda i: (0, 0))],
        out_specs=[pl.BlockSpec((_BK, 128), lambda i: (i, 0)),
                   pl.BlockSpec((_BK, VOCAB), lambda i: (i, 0)),
                   pl.BlockSpec((_BK, 16), lambda i: (i, 0)),
                   pl.BlockSpec((_BK, 16), lambda i: (i, 0))],
        out_shape=[jax.ShapeDtypeStruct((N_Y, 128), _f32),
                   jax.ShapeDtypeStruct((N_Y, VOCAB), _f32),
                   jax.ShapeDtypeStruct((N_Y, 16), _f32),
                   jax.ShapeDtypeStruct((N_Y, 16), _f32)],
    )(ylin, agg, ctx, yb, bias, wz, bz, wgs, wgd, bgs)


_BKE = 2560


def _edge_softmax_tc(gs2, gd2):
    grid = E // _BKE
    bk8 = _BKE // 8

    def kfn(gs_ref, gd_ref, out_ref):
        sv = gs_ref[...] + gd_ref[...]
        m = jnp.max(sv, axis=1, keepdims=True)
        ex = jnp.exp(sv - m)
        gi = lax.broadcasted_iota(jnp.int32, (128, 128), 0) // 16
        gj = lax.broadcasted_iota(jnp.int32, (128, 128), 1) // 16
        gmat = (gi == gj).astype(_f32)
        sums = jnp.dot(ex, gmat, preferred_element_type=_f32)
        out_ref[...] = sv - m - jnp.log(sums)

    return pl.pallas_call(
        kfn, grid=(grid,),
        in_specs=[pl.BlockSpec((bk8, 128), lambda i: (i, 0)),
                  pl.BlockSpec((bk8, 128), lambda i: (i, 0))],
        out_specs=pl.BlockSpec((bk8, 128), lambda i: (i, 0)),
        out_shape=jax.ShapeDtypeStruct((E // 8, 128), _f32),
    )(gs2, gd2)


def kernel(x, x_batch, tgt_y, tgt_edge_index, tgt_edge_type, tgt_y_batch,
           embed_table,
           l1_Ws, l1_b, l1_We, l1_Wc,
           l2_Ws, l2_b, l2_We, l2_Wc,
           l3_Ws, l3_b, l3_We, l3_Wc,
           Wz, bz, Wg, bg):
    src3 = tgt_edge_index[0].reshape(NW, NPW, CH)
    dst3 = tgt_edge_index[1].reshape(NW, NPW, CH)
    dst1 = tgt_edge_index[1]
    flat1 = _flat_idx_tc(tgt_edge_index[0].reshape(E // 128, 128),
                         tgt_edge_type.reshape(E // 128, 128)).reshape(E)
    yb = tgt_y_batch.reshape(N_Y, 1)
    zeros_z = jnp.zeros((ZB, 128), _f32)

    def wcat(ws, we):
        return jnp.concatenate([ws, we[0], we[1], we[2], we[3]], axis=1)

    neg = jnp.full((7,), -1e30, _f32)
    wgs = jnp.pad(Wg[:EMB], ((0, 0), (0, 7)))
    wgd = jnp.pad(Wg[EMB:], ((0, 0), (0, 7)))
    bgs = jnp.concatenate([bg, neg]).reshape(1, 16)

    g = _emb_gather(embed_table, tgt_y.T.reshape(-1))
    g = g.reshape(4, N_Y, 128)

    ctxp = _ctx_tc(x, x_batch.reshape(N_X, 1), jnp.stack([l1_Wc, l2_Wc, l3_Wc]))

    ylin1, hall1 = _pre1_tc(g, wcat(l1_Ws, l1_We))
    agg1 = _edge_agg(hall1.reshape(4 * N_Y, 128), flat1, dst1, zeros_z)
    ylin2, hall2 = _mid_tc(ylin1, agg1, ctxp[0], yb, l1_b.reshape(1, 128),
                           wcat(l2_Ws, l2_We))
    agg2 = _edge_agg(hall2.reshape(4 * N_Y, 128), flat1, dst1, zeros_z)
    ylin3, hall3 = _mid_tc(ylin2, agg2, ctxp[1], yb, l2_b.reshape(1, 128),
                           wcat(l3_Ws, l3_We))
    agg3 = _edge_agg(hall3.reshape(4 * N_Y, 128), flat1, dst1, zeros_z)

    y, y_pred, sp, dp = _final_tc(ylin3, agg3, ctxp[2], yb,
                                  l3_b.reshape(1, 128), Wz,
                                  bz.reshape(1, VOCAB), wgs, wgd, bgs)

    gs, gd = _head_gather(sp, dp, src3, dst3)
    packed = _edge_softmax_tc(gs.reshape(E // 8, 128),
                              gd.reshape(E // 8, 128))
    y_edge_pred = packed.reshape(E, 16)[:, :R_EDGE]

    return (y, tgt_edge_index, tgt_edge_type, y_pred, y_edge_pred)


# speedup vs baseline: 25.8324x; 1.0407x over previous
import functools

import jax
import jax.numpy as jnp
from jax import lax
from jax.experimental import pallas as pl
from jax.experimental.pallas import tpu as pltpu
from jax.experimental.pallas import tpu_sc as plsc

N_X = 20000
N_Y = 10000
E = 320000
B = 16
F_SIZE = 128
H_SIZE = 128
EMB = 128
VOCAB = 1000
T_EDGE = 4
R_EDGE = 9

NC, NS = 2, 16
NW = NC * NS
CH = 80
CHA = 40
ZB = 80
NZB = N_Y // ZB

_f32 = jnp.float32


def _sc_mesh():
    return plsc.VectorSubcoreMesh(core_axis_name="c", subcore_axis_name="s")


def _emb_gather(table, idx):
    n_total = idx.shape[0]
    nch = n_total // CH

    @functools.partial(
        pl.kernel, mesh=_sc_mesh(),
        out_type=jax.ShapeDtypeStruct((n_total, 128), _f32),
        scratch_types=[
            pltpu.VMEM((CH,), jnp.int32),
            pltpu.VMEM((CH, 128), _f32),
            pltpu.SemaphoreType.DMA,
        ])
    def k(table_h, idx_h, out_h, idx_v, rows_v, sem):
        c = lax.axis_index("c")
        s = lax.axis_index("s")
        w = s * NC + c
        n = nch // NW + jnp.where(w < (nch % NW), 1, 0)

        def body(t, carry):
            ch = w + NW * t
            base = pl.multiple_of(ch * CH, 8)
            pltpu.sync_copy(idx_h.at[pl.ds(base, CH)], idx_v)
            pltpu.async_copy(table_h.at[idx_v], rows_v, sem).wait()
            pltpu.sync_copy(rows_v, out_h.at[pl.ds(base, CH)])
            return carry

        lax.fori_loop(0, n, body, 0)

    return k(table, idx)


NPW = (E // CH) // NW
NPA = (E // CHA) // NW
NPC = NPA // 2


def _edge_agg(hall, flat_idx, dst_idx, zeros_z):
    @functools.partial(
        pl.kernel, mesh=_sc_mesh(),
        compiler_params=pltpu.CompilerParams(use_tc_tiling_on_sc=False),
        out_type=jax.ShapeDtypeStruct((NC, N_Y, 128), _f32),
        scratch_types=[
            [pltpu.VMEM((4, CHA), jnp.int32)] * 2,
            [pltpu.VMEM((4, CHA), jnp.int32)] * 2,
            [pltpu.VMEM((CHA, 128), _f32)] * 4,
            pltpu.VMEM((ZB, 128), _f32),
            pltpu.VMEM_SHARED((N_Y, 128), _f32),
            [[pltpu.SemaphoreType.DMA] * 4] * 2,
            [[pltpu.SemaphoreType.DMA] * 2] * 2,
            [[pltpu.SemaphoreType.DMA] * 2] * 2,
        ])
    def k(hall_h, flat_h, dst_h, z_h, out_h,
          fidxs, didxs, rowbufs, zbuf, agg_s, isems2, gsems2, ssems2):
        c = lax.axis_index("c")
        s = lax.axis_index("s")
        w = s * NC + c

        class Chain:
            def __init__(self, cid):
                self.fidx = fidxs[cid]
                self.didx = didxs[cid]
                self.rows = rowbufs[2 * cid:2 * cid + 2]
                self.isems = isems2[cid]
                self.gsems = gsems2[cid]
                self.ssems = ssems2[cid]
                self.base0 = w * (NPA * CHA) + cid * (NPC * CHA)

        chains = (Chain(0), Chain(1))

        def ibase(ch, j):
            return pl.multiple_of(ch.base0 + j * CHA, 8)

        def prefetch(ch, j, q):
            @pl.when(j < NPC)
            def _():
                sl = pl.ds(ibase(ch, j), CHA)
                pltpu.async_copy(flat_h.at[sl], ch.fidx.at[q], ch.isems[q])
                pltpu.async_copy(dst_h.at[sl], ch.didx.at[q], ch.isems[q])

        def iwait(ch, j, q):
            sl = pl.ds(ibase(ch, j), CHA)
            pltpu.make_async_copy(flat_h.at[sl], ch.fidx.at[q],
                                  ch.isems[q]).wait()
            pltpu.make_async_copy(dst_h.at[sl], ch.didx.at[q],
                                  ch.isems[q]).wait()

        for ch in chains:
            for q in range(4):
                prefetch(ch, q, q)

        pltpu.sync_copy(z_h, zbuf)
        nzb_s = NZB // NS + jnp.where(s < (NZB % NS), 1, 0)

        def zbody(t, carry):
            off = pl.multiple_of((s + NS * t) * ZB, 8)
            pltpu.sync_copy(zbuf, agg_s.at[pl.ds(off, ZB)])
            return carry

        lax.fori_loop(0, nzb_s, zbody, 0)
        plsc.subcore_barrier()

        for ch in chains:
            iwait(ch, 0, 0)
            pltpu.async_copy(hall_h.at[ch.fidx.at[0]], ch.rows[0],
                             ch.gsems[0])

        def step(ch, j, ph):
            b = ph % 2
            bn = (ph + 1) % 2
            qn = (ph + 1) % 4
            qp = (ph - 1) % 4

            @pl.when(j >= 1)
            def _():
                pltpu.make_async_copy(
                    ch.rows[bn], agg_s.at[ch.didx.at[qp]],
                    ch.ssems[bn]).wait()
                prefetch(ch, j + 3, qp)

            @pl.when(j + 1 < NPC)
            def _():
                iwait(ch, j + 1, qn)
                pltpu.async_copy(hall_h.at[ch.fidx.at[qn]], ch.rows[bn],
                                 ch.gsems[bn])

            pltpu.make_async_copy(
                hall_h.at[ch.fidx.at[ph]], ch.rows[b], ch.gsems[b]).wait()
            pltpu.async_copy(ch.rows[b], agg_s.at[ch.didx.at[ph]],
                             ch.ssems[b], add=True)

        def body(i, carry):
            for ph in range(4):
                for ch in chains:
                    step(ch, 4 * i + ph, ph)
            return carry

        lax.fori_loop(0, NPC // 4, body, 0)
        for ch in chains:
            step(ch, NPC - 1, (NPC - 1) % 4)
        for ch in chains:
            pltpu.make_async_copy(
                ch.rows[(NPC - 1) % 2],
                agg_s.at[ch.didx.at[(NPC - 1) % 4]],
                ch.ssems[(NPC - 1) % 2]).wait()

        plsc.subcore_barrier()

        def obody(t, carry):
            off = pl.multiple_of((s + NS * t) * ZB, 8)
            sl = pl.ds(off, ZB)
            pltpu.sync_copy(agg_s.at[sl], out_h.at[c].at[sl])
            return carry

        lax.fori_loop(0, nzb_s, obody, 0)

    return k(hall, flat_idx, dst_idx, zeros_z)


def _head_gather(sp, dp, src3, dst3):
    @functools.partial(
        pl.kernel, mesh=_sc_mesh(),
        compiler_params=pltpu.CompilerParams(use_tc_tiling_on_sc=False),
        out_type=(jax.ShapeDtypeStruct((E, 16), _f32),
                  jax.ShapeDtypeStruct((E, 16), _f32)),
        scratch_types=[
            pltpu.VMEM((NPW, CH), jnp.int32),
            pltpu.VMEM((NPW, CH), jnp.int32),
            pltpu.VMEM((CH, 16), _f32),
            pltpu.VMEM((CH, 16), _f32),
            pltpu.VMEM((CH, 16), _f32),
            pltpu.VMEM((CH, 16), _f32),
            pltpu.SemaphoreType.DMA,
            pltpu.SemaphoreType.DMA,
            pltpu.SemaphoreType.DMA,
            pltpu.SemaphoreType.DMA,
            pltpu.SemaphoreType.DMA,
            pltpu.SemaphoreType.DMA,
            pltpu.SemaphoreType.DMA,
            pltpu.SemaphoreType.DMA,
            pltpu.SemaphoreType.DMA,
        ])
    def k(sp_h, dp_h, src_h, dst_h, gs_h, gd_h, si_v, di_v,
          a0, a1, b0, b1, ga0, ga1, gb0, gb1, wa0, wa1, wb0, wb1, msem):
        c = lax.axis_index("c")
        s = lax.axis_index("s")
        w = s * NC + c

        pltpu.async_copy(src_h.at[w], si_v, msem)
        pltpu.async_copy(dst_h.at[w], di_v, msem)
        pltpu.make_async_copy(src_h.at[w], si_v, msem).wait()
        pltpu.make_async_copy(dst_h.at[w], di_v, msem).wait()

        bufs = ((a0, ga0, wa0, b0, gb0, wb0), (a1, ga1, wa1, b1, gb1, wb1))

        def gissue(j, av, gas, bv, gbs):
            pltpu.async_copy(sp_h.at[si_v.at[j]], av, gas)
            pltpu.async_copy(dp_h.at[di_v.at[j]], bv, gbs)

        gissue(0, bufs[0][0], bufs[0][1], bufs[0][3], bufs[0][4])
        gissue(1, bufs[1][0], bufs[1][1], bufs[1][3], bufs[1][4])

        def step(j, av, gas, was, bv, gbs, wbs):
            base = pl.multiple_of((w * NPW + j) * CH, 8)
            osl = pl.ds(base, CH)
            pltpu.make_async_copy(sp_h.at[si_v.at[j]], av, gas).wait()
            pltpu.async_copy(av, gs_h.at[osl], was)
            pltpu.make_async_copy(dp_h.at[di_v.at[j]], bv, gbs).wait()
            pltpu.async_copy(bv, gd_h.at[osl], wbs)
            pltpu.make_async_copy(av, gs_h.at[osl], was).wait()
            pltpu.make_async_copy(bv, gd_h.at[osl], wbs).wait()

            @pl.when(j + 2 < NPW)
            def _():
                gissue(j + 2, av, gas, bv, gbs)

        def body(i, carry):
            for b, bb in enumerate(bufs):
                step(2 * i + b, *bb)
            return carry

        lax.fori_loop(0, NPW // 2, body, 0)
        step(NPW - 1, *bufs[(NPW - 1) % 2])

    return k(sp, dp, src3, dst3)


_BK = 400


def _flat_idx_tc(src2, typ2):
    def kfn(s_ref, t_ref, o_ref):
        o_ref[...] = t_ref[...] * N_Y + s_ref[...]

    return pl.pallas_call(
        kfn,
        out_shape=jax.ShapeDtypeStruct((E // 128, 128), jnp.int32),
    )(src2, typ2)


def _ctx_tc(x, x_batch, wc3):
    grid = N_X // _BK

    def kfn(xb_ref, x_ref, wc_ref, out_ref, acc, cnt):
        i = pl.program_id(0)

        @pl.when(i == 0)
        def _():
            acc[...] = jnp.zeros_like(acc)
            cnt[...] = jnp.zeros_like(cnt)

        oh = (xb_ref[...] == lax.broadcasted_iota(jnp.int32, (_BK, B), 1)
              ).astype(_f32)
        acc[...] += lax.dot_general(oh, x_ref[...], (((0,), (0,)), ((), ())),
                                    preferred_element_type=_f32)
        cnt[...] += lax.dot_general(oh, jnp.ones((_BK, 128), _f32),
                                    (((0,), (0,)), ((), ())),
                                    preferred_element_type=_f32)

        @pl.when(i == grid - 1)
        def _():
            ctx = acc[...] / jnp.maximum(cnt[...], 1.0)
            for l in range(3):
                out_ref[l] = jnp.dot(ctx, wc_ref[l],
                                     preferred_element_type=_f32)

    return pl.pallas_call(
        kfn, grid=(grid,),
        in_specs=[pl.BlockSpec((_BK, 1), lambda i: (i, 0)),
                  pl.BlockSpec((_BK, 128), lambda i: (i, 0)),
                  pl.BlockSpec((3, 128, 128), lambda i: (0, 0, 0))],
        out_specs=pl.BlockSpec((3, B, 128), lambda i: (0, 0, 0)),
        out_shape=jax.ShapeDtypeStruct((3, B, 128), _f32),
        scratch_shapes=[pltpu.VMEM((B, 128), _f32),
                        pltpu.VMEM((B, 128), _f32)],
    )(x_batch, x, wc3)


def _pre1_tc(g, wcat):
    grid = N_Y // _BK

    def kfn(g_ref, w_ref, ylin_ref, hall_ref):
        y0 = g_ref[0] + g_ref[1] + g_ref[2] + g_ref[3]
        res = jnp.dot(y0, w_ref[...], preferred_element_type=_f32)
        ylin_ref[...] = res[:, :128]
        for t in range(4):
            hall_ref[t] = res[:, 128 * (t + 1):128 * (t + 2)]

    return pl.pallas_call(
        kfn, grid=(grid,),
        in_specs=[pl.BlockSpec((4, _BK, 128), lambda i: (0, i, 0)),
                  pl.BlockSpec((128, 640), lambda i: (0, 0))],
        out_specs=[pl.BlockSpec((_BK, 128), lambda i: (i, 0)),
                   pl.BlockSpec((4, _BK, 128), lambda i: (0, i, 0))],
        out_shape=[jax.ShapeDtypeStruct((N_Y, 128), _f32),
                   jax.ShapeDtypeStruct((4, N_Y, 128), _f32)],
    )(g, wcat)


def _combine(ylin_ref, agg_ref, ctx_ref, yb_ref, b_ref):
    oh = (yb_ref[...] == lax.broadcasted_iota(jnp.int32, (_BK, B), 1)
          ).astype(_f32)
    ctxg = jnp.dot(oh, ctx_ref[...], preferred_element_type=_f32)
    return jnp.maximum(
        ylin_ref[...] + agg_ref[0] + agg_ref[1] + ctxg + b_ref[...], 0.0)


def _mid_tc(ylin, agg, ctx, yb, bias, wcat):
    grid = N_Y // _BK

    def kfn(ylin_ref, agg_ref, ctx_ref, yb_ref, b_ref, w_ref,
            ylin_o, hall_o):
        y = _combine(ylin_ref, agg_ref, ctx_ref, yb_ref, b_ref)
        res = jnp.dot(y, w_ref[...], preferred_element_type=_f32)
        ylin_o[...] = res[:, :128]
        for t in range(4):
            hall_o[t] = res[:, 128 * (t + 1):128 * (t + 2)]

    return pl.pallas_call(
        kfn, grid=(grid,),
        in_specs=[pl.BlockSpec((_BK, 128), lambda i: (i, 0)),
                  pl.BlockSpec((2, _BK, 128), lambda i: (0, i, 0)),
                  pl.BlockSpec((B, 128), lambda i: (0, 0)),
                  pl.BlockSpec((_BK, 1), lambda i: (i, 0)),
                  pl.BlockSpec((1, 128), lambda i: (0, 0)),
                  pl.BlockSpec((128, 640), lambda i: (0, 0))],
        out_specs=[pl.BlockSpec((_BK, 128), lambda i: (i, 0)),
                   pl.BlockSpec((4, _BK, 128), lambda i: (0, i, 0))],
        out_shape=[jax.ShapeDtypeStruct((N_Y, 128), _f32),
                   jax.ShapeDtypeStruct((4, N_Y, 128), _f32)],
    )(ylin, agg, ctx, yb, bias, wcat)


def _final_tc(ylin, agg, ctx, yb, bias, wz, bz, wgs, wgd, bgs):
    grid = N_Y // _BK

    def kfn(ylin_ref, agg_ref, ctx_ref, yb_ref, b_ref, wz_ref, bz_ref,
            wgs_ref, wgd_ref, bgs_ref, y_o, yp_o, sp_o, dp_o):
        y = _combine(ylin_ref, agg_ref, ctx_ref, yb_ref, b_ref)
        y_o[...] = y
        z = jnp.dot(y, wz_ref[...], preferred_element_type=_f32) + bz_ref[...]
        m = jnp.max(z, axis=1, keepdims=True)
        lse = m + jnp.log(jnp.sum(jnp.exp(z - m), axis=1, keepdims=True))
        yp_o[...] = z - lse
        sp_o[...] = jnp.dot(y, wgs_ref[...],
                            preferred_element_type=_f32) + bgs_ref[...]
        dp_o[...] = jnp.dot(y, wgd_ref[...], preferred_element_type=_f32)

    return pl.pallas_call(
        kfn, grid=(grid,),
        in_specs=[pl.BlockSpec((_BK, 128), lambda i: (i, 0)),
                  pl.BlockSpec((2, _BK, 128), lambda i: (0, i, 0)),
                  pl.BlockSpec((B, 128), lambda i: (0, 0)),
                  pl.BlockSpec((_BK, 1), lambda i: (i, 0)),
                  pl.BlockSpec((1, 128), lambda i: (0, 0)),
                  pl.BlockSpec((128, VOCAB), lambda i: (0, 0)),
                  pl.BlockSpec((1, VOCAB), lambda i: (0, 0)),
                  pl.BlockSpec((128, 16), lambda i: (0, 0)),
                  pl.BlockSpec((128, 16), lambda i: (0, 0)),
                  pl.BlockSpec((1, 16), lambda i: (0, 0))],
        out_specs=[pl.BlockSpec((_BK, 128), lambda i: (i, 0)),
                   pl.BlockSpec((_BK, VOCAB), lambda i: (i, 0)),
                   pl.BlockSpec((_BK, 16), lambda i: (i, 0)),
                   pl.BlockSpec((_BK, 16), lambda i: (i, 0))],
        out_shape=[jax.ShapeDtypeStruct((N_Y, 128), _f32),
                   jax.ShapeDtypeStruct((N_Y, VOCAB), _f32),
                   jax.ShapeDtypeStruct((N_Y, 16), _f32),
                   jax.ShapeDtypeStruct((N_Y, 16), _f32)],
    )(ylin, agg, ctx, yb, bias, wz, bz, wgs, wgd, bgs)


_BKE = 2560


def _edge_softmax_tc(gs2, gd2):
    grid = E // _BKE
    bk8 = _BKE // 8

    def kfn(gs_ref, gd_ref, out_ref):
        sv = gs_ref[...] + gd_ref[...]
        m = jnp.max(sv, axis=1, keepdims=True)
        ex = jnp.exp(sv - m)
        gi = lax.broadcasted_iota(jnp.int32, (128, 128), 0) // 16
        gj = lax.broadcasted_iota(jnp.int32, (128, 128), 1) // 16
        gmat = (gi == gj).astype(_f32)
        sums = jnp.dot(ex, gmat, preferred_element_type=_f32)
        out_ref[...] = sv - m - jnp.log(sums)

    return pl.pallas_call(
        kfn, grid=(grid,),
        in_specs=[pl.BlockSpec((bk8, 128), lambda i: (i, 0)),
                  pl.BlockSpec((bk8, 128), lambda i: (i, 0))],
        out_specs=pl.BlockSpec((bk8, 128), lambda i: (i, 0)),
        out_shape=jax.ShapeDtypeStruct((E // 8, 128), _f32),
    )(gs2, gd2)


def kernel(x, x_batch, tgt_y, tgt_edge_index, tgt_edge_type, tgt_y_batch,
           embed_table,
           l1_Ws, l1_b, l1_We, l1_Wc,
           l2_Ws, l2_b, l2_We, l2_Wc,
           l3_Ws, l3_b, l3_We, l3_Wc,
           Wz, bz, Wg, bg):
    src3 = tgt_edge_index[0].reshape(NW, NPW, CH)
    dst3 = tgt_edge_index[1].reshape(NW, NPW, CH)
    dst1 = tgt_edge_index[1]
    flat1 = _flat_idx_tc(tgt_edge_index[0].reshape(E // 128, 128),
                         tgt_edge_type.reshape(E // 128, 128)).reshape(E)
    yb = tgt_y_batch.reshape(N_Y, 1)
    zeros_z = jnp.zeros((ZB, 128), _f32)

    def wcat(ws, we):
        return jnp.concatenate([ws, we[0], we[1], we[2], we[3]], axis=1)

    neg = jnp.full((7,), -1e30, _f32)
    wgs = jnp.pad(Wg[:EMB], ((0, 0), (0, 7)))
    wgd = jnp.pad(Wg[EMB:], ((0, 0), (0, 7)))
    bgs = jnp.concatenate([bg, neg]).reshape(1, 16)

    g = _emb_gather(embed_table, tgt_y.T.reshape(-1))
    g = g.reshape(4, N_Y, 128)

    ctxp = _ctx_tc(x, x_batch.reshape(N_X, 1), jnp.stack([l1_Wc, l2_Wc, l3_Wc]))

    ylin1, hall1 = _pre1_tc(g, wcat(l1_Ws, l1_We))
    agg1 = _edge_agg(hall1.reshape(4 * N_Y, 128), flat1, dst1, zeros_z)
    ylin2, hall2 = _mid_tc(ylin1, agg1, ctxp[0], yb, l1_b.reshape(1, 128),
                           wcat(l2_Ws, l2_We))
    agg2 = _edge_agg(hall2.reshape(4 * N_Y, 128), flat1, dst1, zeros_z)
    ylin3, hall3 = _mid_tc(ylin2, agg2, ctxp[1], yb, l2_b.reshape(1, 128),
                           wcat(l3_Ws, l3_We))
    agg3 = _edge_agg(hall3.reshape(4 * N_Y, 128), flat1, dst1, zeros_z)

    y, y_pred, sp, dp = _final_tc(ylin3, agg3, ctxp[2], yb,
                                  l3_b.reshape(1, 128), Wz,
                                  bz.reshape(1, VOCAB), wgs, wgd, bgs)

    gs, gd = _head_gather(sp, dp, src3, dst3)
    packed = _edge_softmax_tc(gs.reshape(E // 8, 128),
                              gd.reshape(E // 8, 128))
    y_edge_pred = packed.reshape(E, 16)[:, :R_EDGE]

    return (y, tgt_edge_index, tgt_edge_type, y_pred, y_edge_pred)


# speedup vs baseline: 25.9204x; 1.0034x over previous
import functools

import jax
import jax.numpy as jnp
from jax import lax
from jax.experimental import pallas as pl
from jax.experimental.pallas import tpu as pltpu
from jax.experimental.pallas import tpu_sc as plsc

N_X = 20000
N_Y = 10000
E = 320000
B = 16
F_SIZE = 128
H_SIZE = 128
EMB = 128
VOCAB = 1000
T_EDGE = 4
R_EDGE = 9

NC, NS = 2, 16
NW = NC * NS
CH = 80
CHA = 40
ZB = 80
NZB = N_Y // ZB

_f32 = jnp.float32


def _sc_mesh():
    return plsc.VectorSubcoreMesh(core_axis_name="c", subcore_axis_name="s")


def _emb_gather(table, idx):
    n_total = idx.shape[0]
    nch = n_total // CH

    @functools.partial(
        pl.kernel, mesh=_sc_mesh(),
        out_type=jax.ShapeDtypeStruct((n_total, 128), _f32),
        scratch_types=[
            pltpu.VMEM((CH,), jnp.int32),
            pltpu.VMEM((CH, 128), _f32),
            pltpu.SemaphoreType.DMA,
        ])
    def k(table_h, idx_h, out_h, idx_v, rows_v, sem):
        c = lax.axis_index("c")
        s = lax.axis_index("s")
        w = s * NC + c
        n = nch // NW + jnp.where(w < (nch % NW), 1, 0)

        def body(t, carry):
            ch = w + NW * t
            base = pl.multiple_of(ch * CH, 8)
            pltpu.sync_copy(idx_h.at[pl.ds(base, CH)], idx_v)
            pltpu.async_copy(table_h.at[idx_v], rows_v, sem).wait()
            pltpu.sync_copy(rows_v, out_h.at[pl.ds(base, CH)])
            return carry

        lax.fori_loop(0, n, body, 0)

    return k(table, idx)


NPW = (E // CH) // NW
NPA = (E // CHA) // NW
NPC = NPA // 2


def _edge_agg(hall, flat_idx, dst_idx, zeros_z):
    @functools.partial(
        pl.kernel, mesh=_sc_mesh(),
        compiler_params=pltpu.CompilerParams(use_tc_tiling_on_sc=False),
        out_type=jax.ShapeDtypeStruct((NC, N_Y, 128), _f32),
        scratch_types=[
            [pltpu.VMEM((4, CHA), jnp.int32)] * 2,
            [pltpu.VMEM((4, CHA), jnp.int32)] * 2,
            [pltpu.VMEM((CHA, 128), _f32)] * 4,
            pltpu.VMEM((ZB, 128), _f32),
            pltpu.VMEM_SHARED((N_Y, 128), _f32),
            [[pltpu.SemaphoreType.DMA] * 4] * 2,
            [[pltpu.SemaphoreType.DMA] * 2] * 2,
            [[pltpu.SemaphoreType.DMA] * 2] * 2,
        ])
    def k(hall_h, flat_h, dst_h, z_h, out_h,
          fidxs, didxs, rowbufs, zbuf, agg_s, isems2, gsems2, ssems2):
        c = lax.axis_index("c")
        s = lax.axis_index("s")
        w = s * NC + c

        class Chain:
            def __init__(self, cid):
                self.fidx = fidxs[cid]
                self.didx = didxs[cid]
                self.rows = rowbufs[2 * cid:2 * cid + 2]
                self.isems = isems2[cid]
                self.gsems = gsems2[cid]
                self.ssems = ssems2[cid]
                self.base0 = w * (NPA * CHA) + cid * (NPC * CHA)

        chains = (Chain(0), Chain(1))

        def ibase(ch, j):
            return pl.multiple_of(ch.base0 + j * CHA, 8)

        def prefetch(ch, j, q):
            @pl.when(j < NPC)
            def _():
                sl = pl.ds(ibase(ch, j), CHA)
                pltpu.async_copy(flat_h.at[sl], ch.fidx.at[q], ch.isems[q])
                pltpu.async_copy(dst_h.at[sl], ch.didx.at[q], ch.isems[q])

        def iwait(ch, j, q):
            sl = pl.ds(ibase(ch, j), CHA)
            pltpu.make_async_copy(flat_h.at[sl], ch.fidx.at[q],
                                  ch.isems[q]).wait()
            pltpu.make_async_copy(dst_h.at[sl], ch.didx.at[q],
                                  ch.isems[q]).wait()

        for ch in chains:
            for q in range(4):
                prefetch(ch, q, q)

        pltpu.sync_copy(z_h, zbuf)
        nzb_s = NZB // NS + jnp.where(s < (NZB % NS), 1, 0)

        def zbody(t, carry):
            off = pl.multiple_of((s + NS * t) * ZB, 8)
            pltpu.sync_copy(zbuf, agg_s.at[pl.ds(off, ZB)])
            return carry

        lax.fori_loop(0, nzb_s, zbody, 0)
        plsc.subcore_barrier()

        for ch in chains:
            iwait(ch, 0, 0)
            pltpu.async_copy(hall_h.at[ch.fidx.at[0]], ch.rows[0],
                             ch.gsems[0])

        def step(ch, j, ph):
            b = ph % 2
            bn = (ph + 1) % 2
            qn = (ph + 1) % 4
            qp = (ph - 1) % 4

            @pl.when(j >= 1)
            def _():
                pltpu.make_async_copy(
                    ch.rows[bn], agg_s.at[ch.didx.at[qp]],
                    ch.ssems[bn]).wait()
                prefetch(ch, j + 3, qp)

            @pl.when(j + 1 < NPC)
            def _():
                iwait(ch, j + 1, qn)
                pltpu.async_copy(hall_h.at[ch.fidx.at[qn]], ch.rows[bn],
                                 ch.gsems[bn])

            pltpu.make_async_copy(
                hall_h.at[ch.fidx.at[ph]], ch.rows[b], ch.gsems[b]).wait()
            pltpu.async_copy(ch.rows[b], agg_s.at[ch.didx.at[ph]],
                             ch.ssems[b], add=True)

        def body(i, carry):
            for ph in range(4):
                for ch in chains:
                    step(ch, 4 * i + ph, ph)
            return carry

        lax.fori_loop(0, NPC // 4, body, 0)
        for ch in chains:
            step(ch, NPC - 1, (NPC - 1) % 4)
        for ch in chains:
            pltpu.make_async_copy(
                ch.rows[(NPC - 1) % 2],
                agg_s.at[ch.didx.at[(NPC - 1) % 4]],
                ch.ssems[(NPC - 1) % 2]).wait()

        plsc.subcore_barrier()

        def obody(t, carry):
            off = pl.multiple_of((s + NS * t) * ZB, 8)
            sl = pl.ds(off, ZB)
            pltpu.sync_copy(agg_s.at[sl], out_h.at[c].at[sl])
            return carry

        lax.fori_loop(0, nzb_s, obody, 0)

    return k(hall, flat_idx, dst_idx, zeros_z)


def _head_gather(sp, dp, src3, dst3):
    @functools.partial(
        pl.kernel, mesh=_sc_mesh(),
        compiler_params=pltpu.CompilerParams(use_tc_tiling_on_sc=False),
        out_type=(jax.ShapeDtypeStruct((E, 16), _f32),
                  jax.ShapeDtypeStruct((E, 16), _f32)),
        scratch_types=[
            pltpu.VMEM((NPW, CH), jnp.int32),
            pltpu.VMEM((NPW, CH), jnp.int32),
            pltpu.VMEM((CH, 16), _f32),
            pltpu.VMEM((CH, 16), _f32),
            pltpu.VMEM((CH, 16), _f32),
            pltpu.VMEM((CH, 16), _f32),
            pltpu.SemaphoreType.DMA,
            pltpu.SemaphoreType.DMA,
            pltpu.SemaphoreType.DMA,
            pltpu.SemaphoreType.DMA,
            pltpu.SemaphoreType.DMA,
            pltpu.SemaphoreType.DMA,
            pltpu.SemaphoreType.DMA,
            pltpu.SemaphoreType.DMA,
            pltpu.SemaphoreType.DMA,
        ])
    def k(sp_h, dp_h, src_h, dst_h, gs_h, gd_h, si_v, di_v,
          a0, a1, b0, b1, ga0, ga1, gb0, gb1, wa0, wa1, wb0, wb1, msem):
        c = lax.axis_index("c")
        s = lax.axis_index("s")
        w = s * NC + c

        pltpu.async_copy(src_h.at[w], si_v, msem)
        pltpu.async_copy(dst_h.at[w], di_v, msem)
        pltpu.make_async_copy(src_h.at[w], si_v, msem).wait()
        pltpu.make_async_copy(dst_h.at[w], di_v, msem).wait()

        bufs = ((a0, ga0, wa0, b0, gb0, wb0), (a1, ga1, wa1, b1, gb1, wb1))

        def gissue(j, av, gas, bv, gbs):
            pltpu.async_copy(sp_h.at[si_v.at[j]], av, gas)
            pltpu.async_copy(dp_h.at[di_v.at[j]], bv, gbs)

        gissue(0, bufs[0][0], bufs[0][1], bufs[0][3], bufs[0][4])
        gissue(1, bufs[1][0], bufs[1][1], bufs[1][3], bufs[1][4])

        def step(j, av, gas, was, bv, gbs, wbs):
            base = pl.multiple_of((w * NPW + j) * CH, 8)
            osl = pl.ds(base, CH)
            pltpu.make_async_copy(sp_h.at[si_v.at[j]], av, gas).wait()
            pltpu.async_copy(av, gs_h.at[osl], was)
            pltpu.make_async_copy(dp_h.at[di_v.at[j]], bv, gbs).wait()
            pltpu.async_copy(bv, gd_h.at[osl], wbs)
            pltpu.make_async_copy(av, gs_h.at[osl], was).wait()
            pltpu.make_async_copy(bv, gd_h.at[osl], wbs).wait()

            @pl.when(j + 2 < NPW)
            def _():
                gissue(j + 2, av, gas, bv, gbs)

        def body(i, carry):
            for b, bb in enumerate(bufs):
                step(2 * i + b, *bb)
            return carry

        lax.fori_loop(0, NPW // 2, body, 0)
        step(NPW - 1, *bufs[(NPW - 1) % 2])

    return k(sp, dp, src3, dst3)


_BK = 400


def _flat_idx_tc(src2, typ2):
    def kfn(s_ref, t_ref, o_ref):
        o_ref[...] = t_ref[...] * N_Y + s_ref[...]

    return pl.pallas_call(
        kfn,
        out_shape=jax.ShapeDtypeStruct((E // 128, 128), jnp.int32),
    )(src2, typ2)


def _ctx_tc(x, x_batch, wc3):
    grid = N_X // _BK

    def kfn(xb_ref, x_ref, wc_ref, out_ref, acc, cnt):
        i = pl.program_id(0)

        @pl.when(i == 0)
        def _():
            acc[...] = jnp.zeros_like(acc)
            cnt[...] = jnp.zeros_like(cnt)

        oh = (xb_ref[...] == lax.broadcasted_iota(jnp.int32, (_BK, B), 1)
              ).astype(_f32)
        acc[...] += lax.dot_general(oh, x_ref[...], (((0,), (0,)), ((), ())),
                                    preferred_element_type=_f32)
        cnt[...] += lax.dot_general(oh, jnp.ones((_BK, 128), _f32),
                                    (((0,), (0,)), ((), ())),
                                    preferred_element_type=_f32)

        @pl.when(i == grid - 1)
        def _():
            ctx = acc[...] / jnp.maximum(cnt[...], 1.0)
            for l in range(3):
                out_ref[l] = jnp.dot(ctx, wc_ref[l],
                                     preferred_element_type=_f32)

    return pl.pallas_call(
        kfn, grid=(grid,),
        in_specs=[pl.BlockSpec((_BK, 1), lambda i: (i, 0)),
                  pl.BlockSpec((_BK, 128), lambda i: (i, 0)),
                  pl.BlockSpec((3, 128, 128), lambda i: (0, 0, 0))],
        out_specs=pl.BlockSpec((3, B, 128), lambda i: (0, 0, 0)),
        out_shape=jax.ShapeDtypeStruct((3, B, 128), _f32),
        scratch_shapes=[pltpu.VMEM((B, 128), _f32),
                        pltpu.VMEM((B, 128), _f32)],
    )(x_batch, x, wc3)


def _pre1_tc(g, wcat):
    grid = N_Y // _BK

    def kfn(g_ref, w_ref, ylin_ref, hall_ref):
        y0 = g_ref[0] + g_ref[1] + g_ref[2] + g_ref[3]
        res = jnp.dot(y0, w_ref[...], preferred_element_type=_f32)
        ylin_ref[...] = res[:, :128]
        for t in range(4):
            hall_ref[t] = res[:, 128 * (t + 1):128 * (t + 2)]

    return pl.pallas_call(
        kfn, grid=(grid,),
        in_specs=[pl.BlockSpec((4, _BK, 128), lambda i: (0, i, 0)),
                  pl.BlockSpec((128, 640), lambda i: (0, 0))],
        out_specs=[pl.BlockSpec((_BK, 128), lambda i: (i, 0)),
                   pl.BlockSpec((4, _BK, 128), lambda i: (0, i, 0))],
        out_shape=[jax.ShapeDtypeStruct((N_Y, 128), _f32),
                   jax.ShapeDtypeStruct((4, N_Y, 128), _f32)],
    )(g, wcat)


def _combine(ylin_ref, agg_ref, ctx_ref, yb_ref, b_ref):
    oh = (yb_ref[...] == lax.broadcasted_iota(jnp.int32, (_BK, B), 1)
          ).astype(_f32)
    ctxg = jnp.dot(oh, ctx_ref[...], preferred_element_type=_f32)
    return jnp.maximum(
        ylin_ref[...] + agg_ref[0] + agg_ref[1] + ctxg + b_ref[...], 0.0)


def _mid_tc(ylin, agg, ctx, yb, bias, wcat):
    grid = N_Y // _BK

    def kfn(ylin_ref, agg_ref, ctx_ref, yb_ref, b_ref, w_ref,
            ylin_o, hall_o):
        y = _combine(ylin_ref, agg_ref, ctx_ref, yb_ref, b_ref)
        res = jnp.dot(y, w_ref[...], preferred_element_type=_f32)
        ylin_o[...] = res[:, :128]
        for t in range(4):
            hall_o[t] = res[:, 128 * (t + 1):128 * (t + 2)]

    return pl.pallas_call(
        kfn, grid=(grid,),
        in_specs=[pl.BlockSpec((_BK, 128), lambda i: (i, 0)),
                  pl.BlockSpec((2, _BK, 128), lambda i: (0, i, 0)),
                  pl.BlockSpec((B, 128), lambda i: (0, 0)),
                  pl.BlockSpec((_BK, 1), lambda i: (i, 0)),
                  pl.BlockSpec((1, 128), lambda i: (0, 0)),
                  pl.BlockSpec((128, 640), lambda i: (0, 0))],
        out_specs=[pl.BlockSpec((_BK, 128), lambda i: (i, 0)),
                   pl.BlockSpec((4, _BK, 128), lambda i: (0, i, 0))],
        out_shape=[jax.ShapeDtypeStruct((N_Y, 128), _f32),
                   jax.ShapeDtypeStruct((4, N_Y, 128), _f32)],
    )(ylin, agg, ctx, yb, bias, wcat)


def _combine3_tc(ylin, agg, ctx, yb, bias, wgs, wgd, bgs):
    grid = N_Y // _BK

    def kfn(ylin_ref, agg_ref, ctx_ref, yb_ref, b_ref,
            wgs_ref, wgd_ref, bgs_ref, y_o, sp_o, dp_o):
        y = _combine(ylin_ref, agg_ref, ctx_ref, yb_ref, b_ref)
        y_o[...] = y
        sp_o[...] = jnp.dot(y, wgs_ref[...],
                            preferred_element_type=_f32) + bgs_ref[...]
        dp_o[...] = jnp.dot(y, wgd_ref[...], preferred_element_type=_f32)

    return pl.pallas_call(
        kfn, grid=(grid,),
        in_specs=[pl.BlockSpec((_BK, 128), lambda i: (i, 0)),
                  pl.BlockSpec((2, _BK, 128), lambda i: (0, i, 0)),
                  pl.BlockSpec((B, 128), lambda i: (0, 0)),
                  pl.BlockSpec((_BK, 1), lambda i: (i, 0)),
                  pl.BlockSpec((1, 128), lambda i: (0, 0)),
                  pl.BlockSpec((128, 16), lambda i: (0, 0)),
                  pl.BlockSpec((128, 16), lambda i: (0, 0)),
                  pl.BlockSpec((1, 16), lambda i: (0, 0))],
        out_specs=[pl.BlockSpec((_BK, 128), lambda i: (i, 0)),
                   pl.BlockSpec((_BK, 16), lambda i: (i, 0)),
                   pl.BlockSpec((_BK, 16), lambda i: (i, 0))],
        out_shape=[jax.ShapeDtypeStruct((N_Y, 128), _f32),
                   jax.ShapeDtypeStruct((N_Y, 16), _f32),
                   jax.ShapeDtypeStruct((N_Y, 16), _f32)],
    )(ylin, agg, ctx, yb, bias, wgs, wgd, bgs)


def _vocab_tc(y, wz, bz):
    grid = N_Y // _BK

    def kfn(y_ref, wz_ref, bz_ref, yp_o):
        z = jnp.dot(y_ref[...], wz_ref[...],
                    preferred_element_type=_f32) + bz_ref[...]
        m = jnp.max(z, axis=1, keepdims=True)
        lse = m + jnp.log(jnp.sum(jnp.exp(z - m), axis=1, keepdims=True))
        yp_o[...] = z - lse

    return pl.pallas_call(
        kfn, grid=(grid,),
        in_specs=[pl.BlockSpec((_BK, 128), lambda i: (i, 0)),
                  pl.BlockSpec((128, VOCAB), lambda i: (0, 0)),
                  pl.BlockSpec((1, VOCAB), lambda i: (0, 0))],
        out_specs=pl.BlockSpec((_BK, VOCAB), lambda i: (i, 0)),
        out_shape=jax.ShapeDtypeStruct((N_Y, VOCAB), _f32),
    )(y, wz, bz)


_BKE = 2560


def _edge_softmax_tc(gs2, gd2):
    grid = E // _BKE
    bk8 = _BKE // 8

    def kfn(gs_ref, gd_ref, out_ref):
        sv = gs_ref[...] + gd_ref[...]
        m = jnp.max(sv, axis=1, keepdims=True)
        ex = jnp.exp(sv - m)
        gi = lax.broadcasted_iota(jnp.int32, (128, 128), 0) // 16
        gj = lax.broadcasted_iota(jnp.int32, (128, 128), 1) // 16
        gmat = (gi == gj).astype(_f32)
        sums = jnp.dot(ex, gmat, preferred_element_type=_f32)
        out_ref[...] = sv - m - jnp.log(sums)

    return pl.pallas_call(
        kfn, grid=(grid,),
        in_specs=[pl.BlockSpec((bk8, 128), lambda i: (i, 0)),
                  pl.BlockSpec((bk8, 128), lambda i: (i, 0))],
        out_specs=pl.BlockSpec((bk8, 128), lambda i: (i, 0)),
        out_shape=jax.ShapeDtypeStruct((E // 8, 128), _f32),
    )(gs2, gd2)


def kernel(x, x_batch, tgt_y, tgt_edge_index, tgt_edge_type, tgt_y_batch,
           embed_table,
           l1_Ws, l1_b, l1_We, l1_Wc,
           l2_Ws, l2_b, l2_We, l2_Wc,
           l3_Ws, l3_b, l3_We, l3_Wc,
           Wz, bz, Wg, bg):
    src3 = tgt_edge_index[0].reshape(NW, NPW, CH)
    dst3 = tgt_edge_index[1].reshape(NW, NPW, CH)
    dst1 = tgt_edge_index[1]
    flat1 = _flat_idx_tc(tgt_edge_index[0].reshape(E // 128, 128),
                         tgt_edge_type.reshape(E // 128, 128)).reshape(E)
    yb = tgt_y_batch.reshape(N_Y, 1)
    zeros_z = jnp.zeros((ZB, 128), _f32)

    def wcat(ws, we):
        return jnp.concatenate([ws, we[0], we[1], we[2], we[3]], axis=1)

    neg = jnp.full((7,), -1e30, _f32)
    wgs = jnp.pad(Wg[:EMB], ((0, 0), (0, 7)))
    wgd = jnp.pad(Wg[EMB:], ((0, 0), (0, 7)))
    bgs = jnp.concatenate([bg, neg]).reshape(1, 16)

    g = _emb_gather(embed_table, tgt_y.T.reshape(-1))
    g = g.reshape(4, N_Y, 128)

    ctxp = _ctx_tc(x, x_batch.reshape(N_X, 1), jnp.stack([l1_Wc, l2_Wc, l3_Wc]))

    ylin1, hall1 = _pre1_tc(g, wcat(l1_Ws, l1_We))
    agg1 = _edge_agg(hall1.reshape(4 * N_Y, 128), flat1, dst1, zeros_z)
    ylin2, hall2 = _mid_tc(ylin1, agg1, ctxp[0], yb, l1_b.reshape(1, 128),
                           wcat(l2_Ws, l2_We))
    agg2 = _edge_agg(hall2.reshape(4 * N_Y, 128), flat1, dst1, zeros_z)
    ylin3, hall3 = _mid_tc(ylin2, agg2, ctxp[1], yb, l2_b.reshape(1, 128),
                           wcat(l3_Ws, l3_We))
    agg3 = _edge_agg(hall3.reshape(4 * N_Y, 128), flat1, dst1, zeros_z)

    y, sp, dp = _combine3_tc(ylin3, agg3, ctxp[2], yb,
                             l3_b.reshape(1, 128), wgs, wgd, bgs)

    gs, gd = _head_gather(sp, dp, src3, dst3)
    y_pred = _vocab_tc(y, Wz, bz.reshape(1, VOCAB))
    packed = _edge_softmax_tc(gs.reshape(E // 8, 128),
                              gd.reshape(E // 8, 128))
    y_edge_pred = packed.reshape(E, 16)[:, :R_EDGE]

    return (y, tgt_edge_index, tgt_edge_type, y_pred, y_edge_pred)


# speedup vs baseline: 27.8176x; 1.0732x over previous
import functools

import jax
import jax.numpy as jnp
from jax import lax
from jax.experimental import pallas as pl
from jax.experimental.pallas import tpu as pltpu
from jax.experimental.pallas import tpu_sc as plsc

N_X = 20000
N_Y = 10000
E = 320000
B = 16
F_SIZE = 128
H_SIZE = 128
EMB = 128
VOCAB = 1000
T_EDGE = 4
R_EDGE = 9

NC, NS = 2, 16
NW = NC * NS
CH = 80
CHA = 40
ZB = 80
NZB = N_Y // ZB

_f32 = jnp.float32


def _sc_mesh():
    return plsc.VectorSubcoreMesh(core_axis_name="c", subcore_axis_name="s")


def _emb_gather(table, idx):
    n_total = idx.shape[0]
    nch = n_total // CH

    @functools.partial(
        pl.kernel, mesh=_sc_mesh(),
        out_type=jax.ShapeDtypeStruct((n_total, 128), _f32),
        scratch_types=[
            pltpu.VMEM((CH,), jnp.int32),
            pltpu.VMEM((CH, 128), _f32),
            pltpu.SemaphoreType.DMA,
        ])
    def k(table_h, idx_h, out_h, idx_v, rows_v, sem):
        c = lax.axis_index("c")
        s = lax.axis_index("s")
        w = s * NC + c
        n = nch // NW + jnp.where(w < (nch % NW), 1, 0)

        def body(t, carry):
            ch = w + NW * t
            base = pl.multiple_of(ch * CH, 8)
            pltpu.sync_copy(idx_h.at[pl.ds(base, CH)], idx_v)
            pltpu.async_copy(table_h.at[idx_v], rows_v, sem).wait()
            pltpu.sync_copy(rows_v, out_h.at[pl.ds(base, CH)])
            return carry

        lax.fori_loop(0, n, body, 0)

    return k(table, idx)


NPW = (E // CH) // NW
NPA = (E // CHA) // NW
NPC = NPA // 2


def _edge_agg(hall, flat_idx, dst_idx, zeros_z):
    @functools.partial(
        pl.kernel, mesh=_sc_mesh(),
        compiler_params=pltpu.CompilerParams(use_tc_tiling_on_sc=False),
        out_type=jax.ShapeDtypeStruct((NC, N_Y, 128), _f32),
        scratch_types=[
            [pltpu.VMEM((4, CHA), jnp.int32)] * 2,
            [pltpu.VMEM((4, CHA), jnp.int32)] * 2,
            [pltpu.VMEM((CHA, 128), _f32)] * 4,
            pltpu.VMEM((ZB, 128), _f32),
            pltpu.VMEM_SHARED((N_Y, 128), _f32),
            [[pltpu.SemaphoreType.DMA] * 4] * 2,
            [[pltpu.SemaphoreType.DMA] * 2] * 2,
            [[pltpu.SemaphoreType.DMA] * 2] * 2,
        ])
    def k(hall_h, flat_h, dst_h, z_h, out_h,
          fidxs, didxs, rowbufs, zbuf, agg_s, isems2, gsems2, ssems2):
        c = lax.axis_index("c")
        s = lax.axis_index("s")
        w = s * NC + c

        class Chain:
            def __init__(self, cid):
                self.fidx = fidxs[cid]
                self.didx = didxs[cid]
                self.rows = rowbufs[2 * cid:2 * cid + 2]
                self.isems = isems2[cid]
                self.gsems = gsems2[cid]
                self.ssems = ssems2[cid]
                self.base0 = w * (NPA * CHA) + cid * (NPC * CHA)

        chains = (Chain(0), Chain(1))

        def ibase(ch, j):
            return pl.multiple_of(ch.base0 + j * CHA, 8)

        def prefetch(ch, j, q):
            @pl.when(j < NPC)
            def _():
                sl = pl.ds(ibase(ch, j), CHA)
                pltpu.async_copy(flat_h.at[sl], ch.fidx.at[q], ch.isems[q])
                pltpu.async_copy(dst_h.at[sl], ch.didx.at[q], ch.isems[q])

        def iwait(ch, j, q):
            sl = pl.ds(ibase(ch, j), CHA)
            pltpu.make_async_copy(flat_h.at[sl], ch.fidx.at[q],
                                  ch.isems[q]).wait()
            pltpu.make_async_copy(dst_h.at[sl], ch.didx.at[q],
                                  ch.isems[q]).wait()

        for ch in chains:
            for q in range(4):
                prefetch(ch, q, q)

        pltpu.sync_copy(z_h, zbuf)
        nzb_s = NZB // NS + jnp.where(s < (NZB % NS), 1, 0)

        def zbody(t, carry):
            off = pl.multiple_of((s + NS * t) * ZB, 8)
            pltpu.sync_copy(zbuf, agg_s.at[pl.ds(off, ZB)])
            return carry

        lax.fori_loop(0, nzb_s, zbody, 0)
        plsc.subcore_barrier()

        for ch in chains:
            iwait(ch, 0, 0)
            pltpu.async_copy(hall_h.at[ch.fidx.at[0]], ch.rows[0],
                             ch.gsems[0])

        def step(ch, j, ph):
            b = ph % 2
            bn = (ph + 1) % 2
            qn = (ph + 1) % 4
            qp = (ph - 1) % 4

            @pl.when(j >= 1)
            def _():
                pltpu.make_async_copy(
                    ch.rows[bn], agg_s.at[ch.didx.at[qp]],
                    ch.ssems[bn]).wait()
                prefetch(ch, j + 3, qp)

            @pl.when(j + 1 < NPC)
            def _():
                iwait(ch, j + 1, qn)
                pltpu.async_copy(hall_h.at[ch.fidx.at[qn]], ch.rows[bn],
                                 ch.gsems[bn])

            pltpu.make_async_copy(
                hall_h.at[ch.fidx.at[ph]], ch.rows[b], ch.gsems[b]).wait()
            pltpu.async_copy(ch.rows[b], agg_s.at[ch.didx.at[ph]],
                             ch.ssems[b], add=True)

        def body(i, carry):
            for ph in range(4):
                for ch in chains:
                    step(ch, 4 * i + ph, ph)
            return carry

        lax.fori_loop(0, NPC // 4, body, 0)
        for ch in chains:
            step(ch, NPC - 1, (NPC - 1) % 4)
        for ch in chains:
            pltpu.make_async_copy(
                ch.rows[(NPC - 1) % 2],
                agg_s.at[ch.didx.at[(NPC - 1) % 4]],
                ch.ssems[(NPC - 1) % 2]).wait()

        plsc.subcore_barrier()

        def obody(t, carry):
            off = pl.multiple_of((s + NS * t) * ZB, 8)
            sl = pl.ds(off, ZB)
            pltpu.sync_copy(agg_s.at[sl], out_h.at[c].at[sl])
            return carry

        lax.fori_loop(0, nzb_s, obody, 0)

    return k(hall, flat_idx, dst_idx, zeros_z)


CHH = 128
NPH = 10000 // CHH
NPH1 = NPH + 1


def _head_gather(sp, dp, src3, dst3):
    @functools.partial(
        pl.kernel, mesh=_sc_mesh(),
        compiler_params=pltpu.CompilerParams(use_tc_tiling_on_sc=False),
        out_type=(jax.ShapeDtypeStruct((E, 16), _f32),
                  jax.ShapeDtypeStruct((E, 16), _f32)),
        scratch_types=[
            pltpu.VMEM((NPH1, CHH), jnp.int32),
            pltpu.VMEM((NPH1, CHH), jnp.int32),
            [pltpu.VMEM((CHH, 16), _f32)] * 4,
            [pltpu.SemaphoreType.DMA] * 4,
            [pltpu.SemaphoreType.DMA] * 4,
            pltpu.SemaphoreType.DMA,
        ])
    def k(sp_h, dp_h, src_h, dst_h, gs_h, gd_h, si_v, di_v,
          rbufs, gsems, wsems, msem):
        c = lax.axis_index("c")
        s = lax.axis_index("s")
        w = s * NC + c

        pltpu.async_copy(src_h.at[w], si_v, msem)
        pltpu.async_copy(dst_h.at[w], di_v, msem)
        pltpu.make_async_copy(src_h.at[w], si_v, msem).wait()
        pltpu.make_async_copy(dst_h.at[w], di_v, msem).wait()

        bufs = ((rbufs[0], gsems[0], wsems[0], rbufs[2], gsems[2], wsems[2]),
                (rbufs[1], gsems[1], wsems[1], rbufs[3], gsems[3], wsems[3]))

        def gissue(j, av, gas, bv, gbs):
            pltpu.async_copy(sp_h.at[si_v.at[j]], av, gas)
            pltpu.async_copy(dp_h.at[di_v.at[j]], bv, gbs)

        gissue(0, bufs[0][0], bufs[0][1], bufs[0][3], bufs[0][4])
        gissue(1, bufs[1][0], bufs[1][1], bufs[1][3], bufs[1][4])

        def step(j, av, gas, was, bv, gbs, wbs):
            base = pl.multiple_of(w * 10000 + j * CHH, 8)
            osl = pl.ds(base, CHH)
            pltpu.make_async_copy(sp_h.at[si_v.at[j]], av, gas).wait()
            pltpu.async_copy(av, gs_h.at[osl], was)
            pltpu.make_async_copy(dp_h.at[di_v.at[j]], bv, gbs).wait()
            pltpu.async_copy(bv, gd_h.at[osl], wbs)
            pltpu.make_async_copy(av, gs_h.at[osl], was).wait()
            pltpu.make_async_copy(bv, gd_h.at[osl], wbs).wait()

            @pl.when(j + 2 < NPH1)
            def _():
                gissue(j + 2, av, gas, bv, gbs)

        def body(i, carry):
            for b, bb in enumerate(bufs):
                step(2 * i + b, *bb)
            return carry

        lax.fori_loop(0, NPH // 2, body, 0)
        av, gas, was, bv, gbs, wbs = bufs[NPH % 2]
        tbase = pl.multiple_of(w * 10000 + NPH * CHH, 8)
        tsl = pl.ds(tbase, 16)
        pltpu.make_async_copy(sp_h.at[si_v.at[NPH]], av, gas).wait()
        pltpu.make_async_copy(dp_h.at[di_v.at[NPH]], bv, gbs).wait()
        pltpu.async_copy(av.at[pl.ds(0, 16)], gs_h.at[tsl], was)
        pltpu.async_copy(bv.at[pl.ds(0, 16)], gd_h.at[tsl], wbs)
        pltpu.make_async_copy(av.at[pl.ds(0, 16)], gs_h.at[tsl], was).wait()
        pltpu.make_async_copy(bv.at[pl.ds(0, 16)], gd_h.at[tsl], wbs).wait()

    return k(sp, dp, src3, dst3)


_BK = 400


def _flat_idx_tc(src2, typ2):
    def kfn(s_ref, t_ref, o_ref):
        o_ref[...] = t_ref[...] * N_Y + s_ref[...]

    return pl.pallas_call(
        kfn,
        out_shape=jax.ShapeDtypeStruct((E // 128, 128), jnp.int32),
    )(src2, typ2)


def _ctx_tc(x, x_batch, wc3):
    grid = N_X // _BK

    def kfn(xb_ref, x_ref, wc_ref, out_ref, acc, cnt):
        i = pl.program_id(0)

        @pl.when(i == 0)
        def _():
            acc[...] = jnp.zeros_like(acc)
            cnt[...] = jnp.zeros_like(cnt)

        oh = (xb_ref[...] == lax.broadcasted_iota(jnp.int32, (_BK, B), 1)
              ).astype(_f32)
        acc[...] += lax.dot_general(oh, x_ref[...], (((0,), (0,)), ((), ())),
                                    preferred_element_type=_f32)
        cnt[...] += lax.dot_general(oh, jnp.ones((_BK, 128), _f32),
                                    (((0,), (0,)), ((), ())),
                                    preferred_element_type=_f32)

        @pl.when(i == grid - 1)
        def _():
            ctx = acc[...] / jnp.maximum(cnt[...], 1.0)
            for l in range(3):
                out_ref[l] = jnp.dot(ctx, wc_ref[l],
                                     preferred_element_type=_f32)

    return pl.pallas_call(
        kfn, grid=(grid,),
        in_specs=[pl.BlockSpec((_BK, 1), lambda i: (i, 0)),
                  pl.BlockSpec((_BK, 128), lambda i: (i, 0)),
                  pl.BlockSpec((3, 128, 128), lambda i: (0, 0, 0))],
        out_specs=pl.BlockSpec((3, B, 128), lambda i: (0, 0, 0)),
        out_shape=jax.ShapeDtypeStruct((3, B, 128), _f32),
        scratch_shapes=[pltpu.VMEM((B, 128), _f32),
                        pltpu.VMEM((B, 128), _f32)],
    )(x_batch, x, wc3)


def _pre1_tc(g, wcat):
    grid = N_Y // _BK

    def kfn(g_ref, w_ref, ylin_ref, hall_ref):
        y0 = g_ref[0] + g_ref[1] + g_ref[2] + g_ref[3]
        res = jnp.dot(y0, w_ref[...], preferred_element_type=_f32)
        ylin_ref[...] = res[:, :128]
        for t in range(4):
            hall_ref[t] = res[:, 128 * (t + 1):128 * (t + 2)]

    return pl.pallas_call(
        kfn, grid=(grid,),
        in_specs=[pl.BlockSpec((4, _BK, 128), lambda i: (0, i, 0)),
                  pl.BlockSpec((128, 640), lambda i: (0, 0))],
        out_specs=[pl.BlockSpec((_BK, 128), lambda i: (i, 0)),
                   pl.BlockSpec((4, _BK, 128), lambda i: (0, i, 0))],
        out_shape=[jax.ShapeDtypeStruct((N_Y, 128), _f32),
                   jax.ShapeDtypeStruct((4, N_Y, 128), _f32)],
    )(g, wcat)


def _combine(ylin_ref, agg_ref, ctx_ref, yb_ref, b_ref):
    oh = (yb_ref[...] == lax.broadcasted_iota(jnp.int32, (_BK, B), 1)
          ).astype(_f32)
    ctxg = jnp.dot(oh, ctx_ref[...], preferred_element_type=_f32)
    return jnp.maximum(
        ylin_ref[...] + agg_ref[0] + agg_ref[1] + ctxg + b_ref[...], 0.0)


def _mid_tc(ylin, agg, ctx, yb, bias, wcat):
    grid = N_Y // _BK

    def kfn(ylin_ref, agg_ref, ctx_ref, yb_ref, b_ref, w_ref,
            ylin_o, hall_o):
        y = _combine(ylin_ref, agg_ref, ctx_ref, yb_ref, b_ref)
        res = jnp.dot(y, w_ref[...], preferred_element_type=_f32)
        ylin_o[...] = res[:, :128]
        for t in range(4):
            hall_o[t] = res[:, 128 * (t + 1):128 * (t + 2)]

    return pl.pallas_call(
        kfn, grid=(grid,),
        in_specs=[pl.BlockSpec((_BK, 128), lambda i: (i, 0)),
                  pl.BlockSpec((2, _BK, 128), lambda i: (0, i, 0)),
                  pl.BlockSpec((B, 128), lambda i: (0, 0)),
                  pl.BlockSpec((_BK, 1), lambda i: (i, 0)),
                  pl.BlockSpec((1, 128), lambda i: (0, 0)),
                  pl.BlockSpec((128, 640), lambda i: (0, 0))],
        out_specs=[pl.BlockSpec((_BK, 128), lambda i: (i, 0)),
                   pl.BlockSpec((4, _BK, 128), lambda i: (0, i, 0))],
        out_shape=[jax.ShapeDtypeStruct((N_Y, 128), _f32),
                   jax.ShapeDtypeStruct((4, N_Y, 128), _f32)],
    )(ylin, agg, ctx, yb, bias, wcat)


def _combine3_tc(ylin, agg, ctx, yb, bias, wgs, wgd, bgs):
    grid = N_Y // _BK

    def kfn(ylin_ref, agg_ref, ctx_ref, yb_ref, b_ref,
            wgs_ref, wgd_ref, bgs_ref, y_o, sp_o, dp_o):
        y = _combine(ylin_ref, agg_ref, ctx_ref, yb_ref, b_ref)
        y_o[...] = y
        sp_o[...] = jnp.dot(y, wgs_ref[...],
                            preferred_element_type=_f32) + bgs_ref[...]
        dp_o[...] = jnp.dot(y, wgd_ref[...], preferred_element_type=_f32)

    return pl.pallas_call(
        kfn, grid=(grid,),
        in_specs=[pl.BlockSpec((_BK, 128), lambda i: (i, 0)),
                  pl.BlockSpec((2, _BK, 128), lambda i: (0, i, 0)),
                  pl.BlockSpec((B, 128), lambda i: (0, 0)),
                  pl.BlockSpec((_BK, 1), lambda i: (i, 0)),
                  pl.BlockSpec((1, 128), lambda i: (0, 0)),
                  pl.BlockSpec((128, 16), lambda i: (0, 0)),
                  pl.BlockSpec((128, 16), lambda i: (0, 0)),
                  pl.BlockSpec((1, 16), lambda i: (0, 0))],
        out_specs=[pl.BlockSpec((_BK, 128), lambda i: (i, 0)),
                   pl.BlockSpec((_BK, 16), lambda i: (i, 0)),
                   pl.BlockSpec((_BK, 16), lambda i: (i, 0))],
        out_shape=[jax.ShapeDtypeStruct((N_Y, 128), _f32),
                   jax.ShapeDtypeStruct((N_Y, 16), _f32),
                   jax.ShapeDtypeStruct((N_Y, 16), _f32)],
    )(ylin, agg, ctx, yb, bias, wgs, wgd, bgs)


def _vocab_tc(y, wz, bz):
    grid = N_Y // _BK

    def kfn(y_ref, wz_ref, bz_ref, yp_o):
        z = jnp.dot(y_ref[...], wz_ref[...],
                    preferred_element_type=_f32) + bz_ref[...]
        m = jnp.max(z, axis=1, keepdims=True)
        lse = m + jnp.log(jnp.sum(jnp.exp(z - m), axis=1, keepdims=True))
        yp_o[...] = z - lse

    return pl.pallas_call(
        kfn, grid=(grid,),
        in_specs=[pl.BlockSpec((_BK, 128), lambda i: (i, 0)),
                  pl.BlockSpec((128, VOCAB), lambda i: (0, 0)),
                  pl.BlockSpec((1, VOCAB), lambda i: (0, 0))],
        out_specs=pl.BlockSpec((_BK, VOCAB), lambda i: (i, 0)),
        out_shape=jax.ShapeDtypeStruct((N_Y, VOCAB), _f32),
    )(y, wz, bz)


_BKE = 16000


def _edge_softmax_tc(gs2, gd2):
    grid = E // _BKE
    bk8 = _BKE // 8

    def kfn(gs_ref, gd_ref, out_ref):
        sv = gs_ref[...] + gd_ref[...]
        m = jnp.max(sv, axis=1, keepdims=True)
        ex = jnp.exp(sv - m)
        gi = lax.broadcasted_iota(jnp.int32, (128, 128), 0) // 16
        gj = lax.broadcasted_iota(jnp.int32, (128, 128), 1) // 16
        gmat = (gi == gj).astype(_f32)
        sums = jnp.dot(ex, gmat, preferred_element_type=_f32)
        out_ref[...] = sv - m - jnp.log(sums)

    return pl.pallas_call(
        kfn, grid=(grid,),
        in_specs=[pl.BlockSpec((bk8, 128), lambda i: (i, 0)),
                  pl.BlockSpec((bk8, 128), lambda i: (i, 0))],
        out_specs=pl.BlockSpec((bk8, 128), lambda i: (i, 0)),
        out_shape=jax.ShapeDtypeStruct((E // 8, 128), _f32),
    )(gs2, gd2)


def kernel(x, x_batch, tgt_y, tgt_edge_index, tgt_edge_type, tgt_y_batch,
           embed_table,
           l1_Ws, l1_b, l1_We, l1_Wc,
           l2_Ws, l2_b, l2_We, l2_Wc,
           l3_Ws, l3_b, l3_We, l3_Wc,
           Wz, bz, Wg, bg):
    def padh(a):
        return jnp.pad(a.reshape(NW, E // NW), ((0, 0), (0, NPH1 * CHH - E // NW))
                       ).reshape(NW, NPH1, CHH)

    src3 = padh(tgt_edge_index[0])
    dst3 = padh(tgt_edge_index[1])
    dst1 = tgt_edge_index[1]
    flat1 = _flat_idx_tc(tgt_edge_index[0].reshape(E // 128, 128),
                         tgt_edge_type.reshape(E // 128, 128)).reshape(E)
    yb = tgt_y_batch.reshape(N_Y, 1)
    zeros_z = jnp.zeros((ZB, 128), _f32)

    def wcat(ws, we):
        return jnp.concatenate([ws, we[0], we[1], we[2], we[3]], axis=1)

    neg = jnp.full((7,), -1e30, _f32)
    wgs = jnp.pad(Wg[:EMB], ((0, 0), (0, 7)))
    wgd = jnp.pad(Wg[EMB:], ((0, 0), (0, 7)))
    bgs = jnp.concatenate([bg, neg]).reshape(1, 16)

    g = _emb_gather(embed_table, tgt_y.T.reshape(-1))
    g = g.reshape(4, N_Y, 128)

    ctxp = _ctx_tc(x, x_batch.reshape(N_X, 1), jnp.stack([l1_Wc, l2_Wc, l3_Wc]))

    ylin1, hall1 = _pre1_tc(g, wcat(l1_Ws, l1_We))
    agg1 = _edge_agg(hall1.reshape(4 * N_Y, 128), flat1, dst1, zeros_z)
    ylin2, hall2 = _mid_tc(ylin1, agg1, ctxp[0], yb, l1_b.reshape(1, 128),
                           wcat(l2_Ws, l2_We))
    agg2 = _edge_agg(hall2.reshape(4 * N_Y, 128), flat1, dst1, zeros_z)
    ylin3, hall3 = _mid_tc(ylin2, agg2, ctxp[1], yb, l2_b.reshape(1, 128),
                           wcat(l3_Ws, l3_We))
    agg3 = _edge_agg(hall3.reshape(4 * N_Y, 128), flat1, dst1, zeros_z)

    y, sp, dp = _combine3_tc(ylin3, agg3, ctxp[2], yb,
                             l3_b.reshape(1, 128), wgs, wgd, bgs)

    gs, gd = _head_gather(sp, dp, src3, dst3)
    y_pred = _vocab_tc(y, Wz, bz.reshape(1, VOCAB))
    packed = _edge_softmax_tc(gs.reshape(E // 8, 128),
                              gd.reshape(E // 8, 128))
    y_edge_pred = packed.reshape(E, 16)[:, :R_EDGE]

    return (y, tgt_edge_index, tgt_edge_type, y_pred, y_edge_pred)


# speedup vs baseline: 29.0066x; 1.0427x over previous
import functools

import jax
import jax.numpy as jnp
from jax import lax
from jax.experimental import pallas as pl
from jax.experimental.pallas import tpu as pltpu
from jax.experimental.pallas import tpu_sc as plsc

N_X = 20000
N_Y = 10000
E = 320000
B = 16
F_SIZE = 128
H_SIZE = 128
EMB = 128
VOCAB = 1000
T_EDGE = 4
R_EDGE = 9

NC, NS = 2, 16
NW = NC * NS
CH = 80
CHA = 40
ZB = 80
NZB = N_Y // ZB

_f32 = jnp.float32


def _sc_mesh():
    return plsc.VectorSubcoreMesh(core_axis_name="c", subcore_axis_name="s")


def _emb_gather(table, idx):
    n_total = idx.shape[0]
    nch = n_total // CH

    @functools.partial(
        pl.kernel, mesh=_sc_mesh(),
        out_type=jax.ShapeDtypeStruct((n_total, 128), _f32),
        scratch_types=[
            pltpu.VMEM((CH,), jnp.int32),
            pltpu.VMEM((CH, 128), _f32),
            pltpu.SemaphoreType.DMA,
        ])
    def k(table_h, idx_h, out_h, idx_v, rows_v, sem):
        c = lax.axis_index("c")
        s = lax.axis_index("s")
        w = s * NC + c
        n = nch // NW + jnp.where(w < (nch % NW), 1, 0)

        def body(t, carry):
            ch = w + NW * t
            base = pl.multiple_of(ch * CH, 8)
            pltpu.sync_copy(idx_h.at[pl.ds(base, CH)], idx_v)
            pltpu.async_copy(table_h.at[idx_v], rows_v, sem).wait()
            pltpu.sync_copy(rows_v, out_h.at[pl.ds(base, CH)])
            return carry

        lax.fori_loop(0, n, body, 0)

    return k(table, idx)


NPW = (E // CH) // NW
NPA = (E // CHA) // NW
NPC = NPA // 2


def _edge_agg(hall, flat_idx, dst_idx, zeros_z):
    @functools.partial(
        pl.kernel, mesh=_sc_mesh(),
        compiler_params=pltpu.CompilerParams(use_tc_tiling_on_sc=False),
        out_type=jax.ShapeDtypeStruct((NC, N_Y, 128), _f32),
        scratch_types=[
            [pltpu.VMEM((4, CHA), jnp.int32)] * 2,
            [pltpu.VMEM((4, CHA), jnp.int32)] * 2,
            [pltpu.VMEM((CHA, 128), _f32)] * 4,
            pltpu.VMEM((ZB, 128), _f32),
            pltpu.VMEM_SHARED((N_Y, 128), _f32),
            [[pltpu.SemaphoreType.DMA] * 4] * 2,
            [[pltpu.SemaphoreType.DMA] * 2] * 2,
            [[pltpu.SemaphoreType.DMA] * 2] * 2,
        ])
    def k(hall_h, flat_h, dst_h, z_h, out_h,
          fidxs, didxs, rowbufs, zbuf, agg_s, isems2, gsems2, ssems2):
        c = lax.axis_index("c")
        s = lax.axis_index("s")
        w = s * NC + c

        class Chain:
            def __init__(self, cid):
                self.fidx = fidxs[cid]
                self.didx = didxs[cid]
                self.rows = rowbufs[2 * cid:2 * cid + 2]
                self.isems = isems2[cid]
                self.gsems = gsems2[cid]
                self.ssems = ssems2[cid]
                self.base0 = w * (NPA * CHA) + cid * (NPC * CHA)

        chains = (Chain(0), Chain(1))

        def ibase(ch, j):
            return pl.multiple_of(ch.base0 + j * CHA, 8)

        def prefetch(ch, j, q):
            @pl.when(j < NPC)
            def _():
                sl = pl.ds(ibase(ch, j), CHA)
                pltpu.async_copy(flat_h.at[sl], ch.fidx.at[q], ch.isems[q])
                pltpu.async_copy(dst_h.at[sl], ch.didx.at[q], ch.isems[q])

        def iwait(ch, j, q):
            sl = pl.ds(ibase(ch, j), CHA)
            pltpu.make_async_copy(flat_h.at[sl], ch.fidx.at[q],
                                  ch.isems[q]).wait()
            pltpu.make_async_copy(dst_h.at[sl], ch.didx.at[q],
                                  ch.isems[q]).wait()

        for ch in chains:
            for q in range(4):
                prefetch(ch, q, q)

        pltpu.sync_copy(z_h, zbuf)
        nzb_s = NZB // NS + jnp.where(s < (NZB % NS), 1, 0)

        def zbody(t, carry):
            off = pl.multiple_of((s + NS * t) * ZB, 8)
            pltpu.sync_copy(zbuf, agg_s.at[pl.ds(off, ZB)])
            return carry

        lax.fori_loop(0, nzb_s, zbody, 0)
        plsc.subcore_barrier()

        for ch in chains:
            iwait(ch, 0, 0)
            pltpu.async_copy(hall_h.at[ch.fidx.at[0]], ch.rows[0],
                             ch.gsems[0])

        def step(ch, j, ph):
            b = ph % 2
            bn = (ph + 1) % 2
            qn = (ph + 1) % 4
            qp = (ph - 1) % 4

            @pl.when(j >= 1)
            def _():
                pltpu.make_async_copy(
                    ch.rows[bn], agg_s.at[ch.didx.at[qp]],
                    ch.ssems[bn]).wait()
                prefetch(ch, j + 3, qp)

            @pl.when(j + 1 < NPC)
            def _():
                iwait(ch, j + 1, qn)
                pltpu.async_copy(hall_h.at[ch.fidx.at[qn]], ch.rows[bn],
                                 ch.gsems[bn])

            pltpu.make_async_copy(
                hall_h.at[ch.fidx.at[ph]], ch.rows[b], ch.gsems[b]).wait()
            pltpu.async_copy(ch.rows[b], agg_s.at[ch.didx.at[ph]],
                             ch.ssems[b], add=True)

        def body(i, carry):
            for ph in range(4):
                for ch in chains:
                    step(ch, 4 * i + ph, ph)
            return carry

        lax.fori_loop(0, NPC // 4, body, 0)
        for ch in chains:
            step(ch, NPC - 1, (NPC - 1) % 4)
        for ch in chains:
            pltpu.make_async_copy(
                ch.rows[(NPC - 1) % 2],
                agg_s.at[ch.didx.at[(NPC - 1) % 4]],
                ch.ssems[(NPC - 1) % 2]).wait()

        plsc.subcore_barrier()

        def obody(t, carry):
            off = pl.multiple_of((s + NS * t) * ZB, 8)
            sl = pl.ds(off, ZB)
            pltpu.sync_copy(agg_s.at[sl], out_h.at[c].at[sl])
            return carry

        lax.fori_loop(0, nzb_s, obody, 0)

    return k(hall, flat_idx, dst_idx, zeros_z)


CHH = 128
NPH = 10000 // CHH
NPH1 = NPH + 1


def _head_gather(sp, dp, src3, dst3):
    @functools.partial(
        pl.kernel, mesh=_sc_mesh(),
        compiler_params=pltpu.CompilerParams(use_tc_tiling_on_sc=False),
        out_type=(jax.ShapeDtypeStruct((E, 16), _f32),
                  jax.ShapeDtypeStruct((E, 16), _f32)),
        scratch_types=[
            pltpu.VMEM((NPH1, CHH), jnp.int32),
            pltpu.VMEM((NPH1, CHH), jnp.int32),
            [pltpu.VMEM((CHH, 16), _f32)] * 4,
            pltpu.VMEM_SHARED((N_Y, 16), _f32),
            pltpu.VMEM_SHARED((N_Y, 16), _f32),
            [pltpu.SemaphoreType.DMA] * 4,
            [pltpu.SemaphoreType.DMA] * 4,
            pltpu.SemaphoreType.DMA,
        ])
    def k(sp_h, dp_h, src_h, dst_h, gs_h, gd_h, si_v, di_v,
          rbufs, sp_s, dp_s, gsems, wsems, msem):
        c = lax.axis_index("c")
        s = lax.axis_index("s")
        w = s * NC + c

        pltpu.async_copy(src_h.at[w], si_v, msem)
        pltpu.async_copy(dst_h.at[w], di_v, msem)

        nzb_s = NZB // NS + jnp.where(s < (NZB % NS), 1, 0)

        def sbody(t, carry):
            off = pl.multiple_of((s + NS * t) * ZB, 8)
            sl = pl.ds(off, ZB)
            pltpu.sync_copy(sp_h.at[sl], sp_s.at[sl])
            pltpu.sync_copy(dp_h.at[sl], dp_s.at[sl])
            return carry

        lax.fori_loop(0, nzb_s, sbody, 0)
        plsc.subcore_barrier()

        pltpu.make_async_copy(src_h.at[w], si_v, msem).wait()
        pltpu.make_async_copy(dst_h.at[w], di_v, msem).wait()

        bufs = ((rbufs[0], gsems[0], wsems[0], rbufs[2], gsems[2], wsems[2]),
                (rbufs[1], gsems[1], wsems[1], rbufs[3], gsems[3], wsems[3]))

        def gissue(j, av, gas, bv, gbs):
            pltpu.async_copy(sp_s.at[si_v.at[j]], av, gas)
            pltpu.async_copy(dp_s.at[di_v.at[j]], bv, gbs)

        gissue(0, bufs[0][0], bufs[0][1], bufs[0][3], bufs[0][4])
        gissue(1, bufs[1][0], bufs[1][1], bufs[1][3], bufs[1][4])

        def step(j, av, gas, was, bv, gbs, wbs):
            base = pl.multiple_of(w * 10000 + j * CHH, 8)
            osl = pl.ds(base, CHH)
            pltpu.make_async_copy(sp_s.at[si_v.at[j]], av, gas).wait()
            pltpu.async_copy(av, gs_h.at[osl], was)
            pltpu.make_async_copy(dp_s.at[di_v.at[j]], bv, gbs).wait()
            pltpu.async_copy(bv, gd_h.at[osl], wbs)
            pltpu.make_async_copy(av, gs_h.at[osl], was).wait()
            pltpu.make_async_copy(bv, gd_h.at[osl], wbs).wait()

            @pl.when(j + 2 < NPH1)
            def _():
                gissue(j + 2, av, gas, bv, gbs)

        def body(i, carry):
            for b, bb in enumerate(bufs):
                step(2 * i + b, *bb)
            return carry

        lax.fori_loop(0, NPH // 2, body, 0)
        av, gas, was, bv, gbs, wbs = bufs[NPH % 2]
        tbase = pl.multiple_of(w * 10000 + NPH * CHH, 8)
        tsl = pl.ds(tbase, 16)
        pltpu.make_async_copy(sp_s.at[si_v.at[NPH]], av, gas).wait()
        pltpu.make_async_copy(dp_s.at[di_v.at[NPH]], bv, gbs).wait()
        pltpu.async_copy(av.at[pl.ds(0, 16)], gs_h.at[tsl], was)
        pltpu.async_copy(bv.at[pl.ds(0, 16)], gd_h.at[tsl], wbs)
        pltpu.make_async_copy(av.at[pl.ds(0, 16)], gs_h.at[tsl], was).wait()
        pltpu.make_async_copy(bv.at[pl.ds(0, 16)], gd_h.at[tsl], wbs).wait()

    return k(sp, dp, src3, dst3)


_BK = 400


def _flat_idx_tc(src2, typ2):
    def kfn(s_ref, t_ref, o_ref):
        o_ref[...] = t_ref[...] * N_Y + s_ref[...]

    return pl.pallas_call(
        kfn,
        out_shape=jax.ShapeDtypeStruct((E // 128, 128), jnp.int32),
    )(src2, typ2)


def _ctx_tc(x, x_batch, wc3):
    grid = N_X // _BK

    def kfn(xb_ref, x_ref, wc_ref, out_ref, acc, cnt):
        i = pl.program_id(0)

        @pl.when(i == 0)
        def _():
            acc[...] = jnp.zeros_like(acc)
            cnt[...] = jnp.zeros_like(cnt)

        oh = (xb_ref[...] == lax.broadcasted_iota(jnp.int32, (_BK, B), 1)
              ).astype(_f32)
        acc[...] += lax.dot_general(oh, x_ref[...], (((0,), (0,)), ((), ())),
                                    preferred_element_type=_f32)
        cnt[...] += lax.dot_general(oh, jnp.ones((_BK, 128), _f32),
                                    (((0,), (0,)), ((), ())),
                                    preferred_element_type=_f32)

        @pl.when(i == grid - 1)
        def _():
            ctx = acc[...] / jnp.maximum(cnt[...], 1.0)
            for l in range(3):
                out_ref[l] = jnp.dot(ctx, wc_ref[l],
                                     preferred_element_type=_f32)

    return pl.pallas_call(
        kfn, grid=(grid,),
        in_specs=[pl.BlockSpec((_BK, 1), lambda i: (i, 0)),
                  pl.BlockSpec((_BK, 128), lambda i: (i, 0)),
                  pl.BlockSpec((3, 128, 128), lambda i: (0, 0, 0))],
        out_specs=pl.BlockSpec((3, B, 128), lambda i: (0, 0, 0)),
        out_shape=jax.ShapeDtypeStruct((3, B, 128), _f32),
        scratch_shapes=[pltpu.VMEM((B, 128), _f32),
                        pltpu.VMEM((B, 128), _f32)],
    )(x_batch, x, wc3)


def _pre1_tc(g, wcat):
    grid = N_Y // _BK

    def kfn(g_ref, w_ref, ylin_ref, hall_ref):
        y0 = g_ref[:, 0] + g_ref[:, 1] + g_ref[:, 2] + g_ref[:, 3]
        res = jnp.dot(y0, w_ref[...], preferred_element_type=_f32)
        ylin_ref[...] = res[:, :128]
        for t in range(4):
            hall_ref[t] = res[:, 128 * (t + 1):128 * (t + 2)]

    return pl.pallas_call(
        kfn, grid=(grid,),
        in_specs=[pl.BlockSpec((_BK, 4, 128), lambda i: (i, 0, 0)),
                  pl.BlockSpec((128, 640), lambda i: (0, 0))],
        out_specs=[pl.BlockSpec((_BK, 128), lambda i: (i, 0)),
                   pl.BlockSpec((4, _BK, 128), lambda i: (0, i, 0))],
        out_shape=[jax.ShapeDtypeStruct((N_Y, 128), _f32),
                   jax.ShapeDtypeStruct((4, N_Y, 128), _f32)],
    )(g, wcat)


def _combine(ylin_ref, agg_ref, ctx_ref, yb_ref, b_ref):
    oh = (yb_ref[...] == lax.broadcasted_iota(jnp.int32, (_BK, B), 1)
          ).astype(_f32)
    ctxg = jnp.dot(oh, ctx_ref[...], preferred_element_type=_f32)
    return jnp.maximum(
        ylin_ref[...] + agg_ref[0] + agg_ref[1] + ctxg + b_ref[...], 0.0)


def _mid_tc(ylin, agg, ctx, yb, bias, wcat):
    grid = N_Y // _BK

    def kfn(ylin_ref, agg_ref, ctx_ref, yb_ref, b_ref, w_ref,
            ylin_o, hall_o):
        y = _combine(ylin_ref, agg_ref, ctx_ref, yb_ref, b_ref)
        res = jnp.dot(y, w_ref[...], preferred_element_type=_f32)
        ylin_o[...] = res[:, :128]
        for t in range(4):
            hall_o[t] = res[:, 128 * (t + 1):128 * (t + 2)]

    return pl.pallas_call(
        kfn, grid=(grid,),
        in_specs=[pl.BlockSpec((_BK, 128), lambda i: (i, 0)),
                  pl.BlockSpec((2, _BK, 128), lambda i: (0, i, 0)),
                  pl.BlockSpec((B, 128), lambda i: (0, 0)),
                  pl.BlockSpec((_BK, 1), lambda i: (i, 0)),
                  pl.BlockSpec((1, 128), lambda i: (0, 0)),
                  pl.BlockSpec((128, 640), lambda i: (0, 0))],
        out_specs=[pl.BlockSpec((_BK, 128), lambda i: (i, 0)),
                   pl.BlockSpec((4, _BK, 128), lambda i: (0, i, 0))],
        out_shape=[jax.ShapeDtypeStruct((N_Y, 128), _f32),
                   jax.ShapeDtypeStruct((4, N_Y, 128), _f32)],
    )(ylin, agg, ctx, yb, bias, wcat)


def _combine3_tc(ylin, agg, ctx, yb, bias, wgs, wgd, bgs):
    grid = N_Y // _BK

    def kfn(ylin_ref, agg_ref, ctx_ref, yb_ref, b_ref,
            wgs_ref, wgd_ref, bgs_ref, y_o, sp_o, dp_o):
        y = _combine(ylin_ref, agg_ref, ctx_ref, yb_ref, b_ref)
        y_o[...] = y
        sp_o[...] = jnp.dot(y, wgs_ref[...],
                            preferred_element_type=_f32) + bgs_ref[...]
        dp_o[...] = jnp.dot(y, wgd_ref[...], preferred_element_type=_f32)

    return pl.pallas_call(
        kfn, grid=(grid,),
        in_specs=[pl.BlockSpec((_BK, 128), lambda i: (i, 0)),
                  pl.BlockSpec((2, _BK, 128), lambda i: (0, i, 0)),
                  pl.BlockSpec((B, 128), lambda i: (0, 0)),
                  pl.BlockSpec((_BK, 1), lambda i: (i, 0)),
                  pl.BlockSpec((1, 128), lambda i: (0, 0)),
                  pl.BlockSpec((128, 16), lambda i: (0, 0)),
                  pl.BlockSpec((128, 16), lambda i: (0, 0)),
                  pl.BlockSpec((1, 16), lambda i: (0, 0))],
        out_specs=[pl.BlockSpec((_BK, 128), lambda i: (i, 0)),
                   pl.BlockSpec((_BK, 16), lambda i: (i, 0)),
                   pl.BlockSpec((_BK, 16), lambda i: (i, 0))],
        out_shape=[jax.ShapeDtypeStruct((N_Y, 128), _f32),
                   jax.ShapeDtypeStruct((N_Y, 16), _f32),
                   jax.ShapeDtypeStruct((N_Y, 16), _f32)],
    )(ylin, agg, ctx, yb, bias, wgs, wgd, bgs)


def _vocab_tc(y, wz, bz):
    grid = N_Y // _BK

    def kfn(y_ref, wz_ref, bz_ref, yp_o):
        z = jnp.dot(y_ref[...], wz_ref[...],
                    preferred_element_type=_f32) + bz_ref[...]
        m = jnp.max(z, axis=1, keepdims=True)
        lse = m + jnp.log(jnp.sum(jnp.exp(z - m), axis=1, keepdims=True))
        yp_o[...] = z - lse

    return pl.pallas_call(
        kfn, grid=(grid,),
        in_specs=[pl.BlockSpec((_BK, 128), lambda i: (i, 0)),
                  pl.BlockSpec((128, VOCAB), lambda i: (0, 0)),
                  pl.BlockSpec((1, VOCAB), lambda i: (0, 0))],
        out_specs=pl.BlockSpec((_BK, VOCAB), lambda i: (i, 0)),
        out_shape=jax.ShapeDtypeStruct((N_Y, VOCAB), _f32),
    )(y, wz, bz)


_BKE = 16000


def _edge_softmax_tc(gs2, gd2):
    grid = E // _BKE
    bk8 = _BKE // 8

    def kfn(gs_ref, gd_ref, out_ref):
        sv = gs_ref[...] + gd_ref[...]
        m = jnp.max(sv, axis=1, keepdims=True)
        ex = jnp.exp(sv - m)
        gi = lax.broadcasted_iota(jnp.int32, (128, 128), 0) // 16
        gj = lax.broadcasted_iota(jnp.int32, (128, 128), 1) // 16
        gmat = (gi == gj).astype(_f32)
        sums = jnp.dot(ex, gmat, preferred_element_type=_f32)
        out_ref[...] = sv - m - jnp.log(sums)

    return pl.pallas_call(
        kfn, grid=(grid,),
        in_specs=[pl.BlockSpec((bk8, 128), lambda i: (i, 0)),
                  pl.BlockSpec((bk8, 128), lambda i: (i, 0))],
        out_specs=pl.BlockSpec((bk8, 128), lambda i: (i, 0)),
        out_shape=jax.ShapeDtypeStruct((E // 8, 128), _f32),
    )(gs2, gd2)


def kernel(x, x_batch, tgt_y, tgt_edge_index, tgt_edge_type, tgt_y_batch,
           embed_table,
           l1_Ws, l1_b, l1_We, l1_Wc,
           l2_Ws, l2_b, l2_We, l2_Wc,
           l3_Ws, l3_b, l3_We, l3_Wc,
           Wz, bz, Wg, bg):
    def padh(a):
        return jnp.pad(a.reshape(NW, E // NW), ((0, 0), (0, NPH1 * CHH - E // NW))
                       ).reshape(NW, NPH1, CHH)

    src3 = padh(tgt_edge_index[0])
    dst3 = padh(tgt_edge_index[1])
    dst1 = tgt_edge_index[1]
    flat1 = _flat_idx_tc(tgt_edge_index[0].reshape(E // 128, 128),
                         tgt_edge_type.reshape(E // 128, 128)).reshape(E)
    yb = tgt_y_batch.reshape(N_Y, 1)
    zeros_z = jnp.zeros((ZB, 128), _f32)

    def wcat(ws, we):
        return jnp.concatenate([ws, we[0], we[1], we[2], we[3]], axis=1)

    neg = jnp.full((7,), -1e30, _f32)
    wgs = jnp.pad(Wg[:EMB], ((0, 0), (0, 7)))
    wgd = jnp.pad(Wg[EMB:], ((0, 0), (0, 7)))
    bgs = jnp.concatenate([bg, neg]).reshape(1, 16)

    g = _emb_gather(embed_table, tgt_y.reshape(-1))
    g = g.reshape(N_Y, 4, 128)

    ctxp = _ctx_tc(x, x_batch.reshape(N_X, 1), jnp.stack([l1_Wc, l2_Wc, l3_Wc]))

    ylin1, hall1 = _pre1_tc(g, wcat(l1_Ws, l1_We))
    agg1 = _edge_agg(hall1.reshape(4 * N_Y, 128), flat1, dst1, zeros_z)
    ylin2, hall2 = _mid_tc(ylin1, agg1, ctxp[0], yb, l1_b.reshape(1, 128),
                           wcat(l2_Ws, l2_We))
    agg2 = _edge_agg(hall2.reshape(4 * N_Y, 128), flat1, dst1, zeros_z)
    ylin3, hall3 = _mid_tc(ylin2, agg2, ctxp[1], yb, l2_b.reshape(1, 128),
                           wcat(l3_Ws, l3_We))
    agg3 = _edge_agg(hall3.reshape(4 * N_Y, 128), flat1, dst1, zeros_z)

    y, sp, dp = _combine3_tc(ylin3, agg3, ctxp[2], yb,
                             l3_b.reshape(1, 128), wgs, wgd, bgs)

    gs, gd = _head_gather(sp, dp, src3, dst3)
    y_pred = _vocab_tc(y, Wz, bz.reshape(1, VOCAB))
    packed = _edge_softmax_tc(gs.reshape(E // 8, 128),
                              gd.reshape(E // 8, 128))
    y_edge_pred = packed.reshape(E, 16)[:, :R_EDGE]

    return (y, tgt_edge_index, tgt_edge_type, y_pred, y_edge_pred)
